# Initial kernel scaffold; baseline (speedup 1.0000x reference)
import jax
import jax.numpy as jnp
from jax.experimental import pallas as pl


def kernel(x, edge_index, edge_attr, mean_vec_x, std_vec_x, mean_vec_edge, std_vec_edge, params):
    raise NotImplementedError("write your pallas kernel here")


# speedup vs baseline: 3.5341x; 3.5341x over previous
import functools

import jax
import jax.numpy as jnp
from jax import lax
from jax.experimental import pallas as pl
from jax.experimental.pallas import tpu as pltpu
from jax.experimental.pallas import tpu_sc as plsc

N = 10000
E = 320000
H = 128
DOUT = 3

BE = 2560
BN = 2000

NC = 2
NS = 16
NW = NC * NS
EPW = E // NW
C = 80
NCHUNK = EPW // C
NPAD = 10240
ROWS_PER_SUB = NPAD // NS


def _ln(u, g, b):
    m = jnp.mean(u, axis=-1, keepdims=True)
    v = jnp.mean((u - m) ** 2, axis=-1, keepdims=True)
    return (u - m) * lax.rsqrt(v + 1e-5) * g + b


def _f32(*shape):
    return jax.ShapeDtypeStruct(shape, jnp.float32)


def _node_enc_body(x, mu, sig, w0, b0, w1, b1, g, b, wa, wb, xe, a, bt):
    xn = (x[...] - mu[...]) / sig[...]
    h = jnp.maximum(jnp.dot(xn, w0[...], preferred_element_type=jnp.float32) + b0[...], 0.0)
    u = jnp.dot(h, w1[...], preferred_element_type=jnp.float32) + b1[...]
    o = _ln(u, g[...], b[...])
    xe[...] = o
    a[...] = jnp.dot(o, wa[...], preferred_element_type=jnp.float32)
    bt[...] = jnp.dot(o, wb[...], preferred_element_type=jnp.float32)


def _edge_enc_body(ea, w0, b0, w1, b1, g, b, out):
    h = jnp.maximum(jnp.dot(ea[...], w0[...], preferred_element_type=jnp.float32) + b0[...], 0.0)
    u = jnp.dot(h, w1[...], preferred_element_type=jnp.float32) + b1[...]
    out[...] = _ln(u, g[...], b[...])


def _edge_mlp_body(ga, gb, ea, w0, b0, w1, b1, g, b, out):
    s = ga[...] + gb[...] + jnp.dot(ea[...], w0[...], preferred_element_type=jnp.float32) + b0[...]
    h = jnp.maximum(s, 0.0)
    u = jnp.dot(h, w1[...], preferred_element_type=jnp.float32) + b1[...]
    out[...] = _ln(u, g[...], b[...]) + ea[...]


def _node_mlp_body(x, a0, a1, w0x, w0a, b0, w1, b1, g, b, wa, wb, xo, a, bt):
    agg = a0[...] + a1[...]
    s = (jnp.dot(x[...], w0x[...], preferred_element_type=jnp.float32)
         + jnp.dot(agg, w0a[...], preferred_element_type=jnp.float32) + b0[...])
    h = jnp.maximum(s, 0.0)
    u = jnp.dot(h, w1[...], preferred_element_type=jnp.float32) + b1[...]
    xn = x[...] + _ln(u, g[...], b[...])
    xo[...] = xn
    a[...] = jnp.dot(xn, wa[...], preferred_element_type=jnp.float32)
    bt[...] = jnp.dot(xn, wb[...], preferred_element_type=jnp.float32)


def _node_dec_body(x, a0, a1, w0x, w0a, b0, w1, b1, g, b, dw0, db0, dw1, db1, out):
    agg = a0[...] + a1[...]
    s = (jnp.dot(x[...], w0x[...], preferred_element_type=jnp.float32)
         + jnp.dot(agg, w0a[...], preferred_element_type=jnp.float32) + b0[...])
    h = jnp.maximum(s, 0.0)
    u = jnp.dot(h, w1[...], preferred_element_type=jnp.float32) + b1[...]
    xn = x[...] + _ln(u, g[...], b[...])
    dh = jnp.maximum(jnp.dot(xn, dw0[...], preferred_element_type=jnp.float32) + db0[...], 0.0)
    out[...] = jnp.dot(dh, dw1[...], preferred_element_type=jnp.float32) + db1[...]


def _row_spec(rows):
    return pl.BlockSpec((rows, H), lambda i: (i, 0))


def _full_spec(shape):
    nd = len(shape)
    return pl.BlockSpec(shape, lambda i: (0,) * nd)


def _tc_call(body, n_out, grid, in_specs, out_rows, interpret=False):
    return pl.pallas_call(
        body,
        grid=(grid,),
        in_specs=in_specs,
        out_specs=[_row_spec(out_rows)] * n_out,
        out_shape=[_f32(grid * out_rows, H)] * n_out,
        interpret=interpret,
    )


def _sc_gather(a_tbl, b_tbl, dst, src):
    mesh = plsc.VectorSubcoreMesh(core_axis_name="c", subcore_axis_name="s")

    @functools.partial(
        pl.kernel,
        out_type=[_f32(E, H), _f32(E, H)],
        mesh=mesh,
        scratch_types=[
            pltpu.VMEM((EPW,), jnp.int32),
            pltpu.VMEM((EPW,), jnp.int32),
            pltpu.VMEM((C, H), jnp.float32),
            pltpu.VMEM((C, H), jnp.float32),
            pltpu.SemaphoreType.DMA,
            pltpu.SemaphoreType.DMA,
        ],
    )
    def k(a_hbm, b_hbm, dst_hbm, src_hbm, ga_hbm, gb_hbm,
          didx, sidx, rows_a, rows_b, sem_a, sem_b):
        wid = lax.axis_index("s") * NC + lax.axis_index("c")
        base = wid * EPW
        pltpu.sync_copy(dst_hbm.at[pl.ds(base, EPW)], didx)
        pltpu.sync_copy(src_hbm.at[pl.ds(base, EPW)], sidx)

        def step(i, _):
            off = i * C
            cpa = pltpu.async_copy(a_hbm.at[didx.at[pl.ds(off, C)]], rows_a, sem_a)
            cpb = pltpu.async_copy(b_hbm.at[sidx.at[pl.ds(off, C)]], rows_b, sem_b)
            cpa.wait()
            cpb.wait()
            pltpu.sync_copy(rows_a, ga_hbm.at[pl.ds(base + off, C)])
            pltpu.sync_copy(rows_b, gb_hbm.at[pl.ds(base + off, C)])
            return 0

        lax.fori_loop(0, NCHUNK, step, 0)

    return k(a_tbl, b_tbl, dst, src)


def _sc_scatter(upd, src, zeros_stripe):
    mesh = plsc.VectorSubcoreMesh(core_axis_name="c", subcore_axis_name="s")

    @functools.partial(
        pl.kernel,
        out_type=_f32(NC, NPAD, H),
        mesh=mesh,
        scratch_types=[
            pltpu.VMEM((EPW,), jnp.int32),
            pltpu.VMEM((C, H), jnp.float32),
            pltpu.VMEM_SHARED((NPAD, H), jnp.float32),
        ],
    )
    def k(upd_hbm, src_hbm, z_hbm, agg_hbm, sidx, buf, shared):
        cid = lax.axis_index("c")
        sid = lax.axis_index("s")
        wid = sid * NC + cid
        base = wid * EPW
        stripe = sid * ROWS_PER_SUB
        pltpu.sync_copy(z_hbm, shared.at[pl.ds(stripe, ROWS_PER_SUB)])
        pltpu.sync_copy(src_hbm.at[pl.ds(base, EPW)], sidx)
        plsc.subcore_barrier()

        def step(i, _):
            off = i * C
            pltpu.sync_copy(upd_hbm.at[pl.ds(base + off, C)], buf)
            pltpu.sync_copy(buf, shared.at[sidx.at[pl.ds(off, C)]], add=True)
            return 0

        lax.fori_loop(0, NCHUNK, step, 0)
        plsc.subcore_barrier()
        pltpu.sync_copy(shared.at[pl.ds(stripe, ROWS_PER_SUB)],
                        agg_hbm.at[cid, pl.ds(stripe, ROWS_PER_SUB)])

    return k(upd, src, zeros_stripe)


def _rowvec(v, width=H):
    return jnp.reshape(v, (1, width)).astype(jnp.float32)


def kernel(x, edge_index, edge_attr, mean_vec_x, std_vec_x, mean_vec_edge,
           std_vec_edge, params):
    dst = edge_index[1].astype(jnp.int32)
    src = edge_index[0].astype(jnp.int32)

    pe = params["node_enc"]
    pee = params["edge_enc"]
    l0 = params["layers"][0]
    l1 = params["layers"][1]
    pd = params["dec"]

    w0e0 = l0["edge_mlp"]["l0"]["w"]
    w0e1 = l1["edge_mlp"]["l0"]["w"]

    sig_e = std_vec_edge.astype(jnp.float32)
    w0_enc = pee["l0"]["w"] / sig_e[:, None]
    b0_enc = pee["l0"]["b"] - mean_vec_edge @ w0_enc
    w0_enc8 = jnp.zeros((8, H), jnp.float32).at[:4].set(w0_enc)
    ea8 = jnp.zeros((E, 8), jnp.float32).at[:, :4].set(edge_attr.astype(jnp.float32))

    full = _full_spec
    node_enc = _tc_call(
        _node_enc_body, 3, N // BN,
        [
            _row_spec(BN), full((1, H)), full((1, H)),
            full((H, H)), full((1, H)), full((H, H)), full((1, H)),
            full((1, H)), full((1, H)), full((H, H)), full((H, H)),
        ],
        BN,
    )
    xe, a_tbl, b_tbl = node_enc(
        x.astype(jnp.float32), _rowvec(mean_vec_x), _rowvec(std_vec_x),
        pe["l0"]["w"], _rowvec(pe["l0"]["b"]), pe["l1"]["w"], _rowvec(pe["l1"]["b"]),
        _rowvec(pe["ln"]["g"]), _rowvec(pe["ln"]["b"]),
        w0e0[:H], w0e0[H:2 * H],
    )

    edge_enc = _tc_call(
        _edge_enc_body, 1, E // BE,
        [
            pl.BlockSpec((BE, 8), lambda i: (i, 0)), pl.BlockSpec((8, H), lambda i: (0, 0)),
            full((1, H)), full((H, H)), full((1, H)), full((1, H)), full((1, H)),
        ],
        BE,
    )
    (ea,) = edge_enc(
        ea8, w0_enc8, _rowvec(b0_enc),
        pee["l1"]["w"], _rowvec(pee["l1"]["b"]),
        _rowvec(pee["ln"]["g"]), _rowvec(pee["ln"]["b"]),
    )

    zeros_stripe = jnp.zeros((ROWS_PER_SUB, H), jnp.float32)

    edge_mlp = _tc_call(
        _edge_mlp_body, 1, E // BE,
        [
            _row_spec(BE), _row_spec(BE), _row_spec(BE),
            full((H, H)), full((1, H)), full((H, H)), full((1, H)),
            full((1, H)), full((1, H)),
        ],
        BE,
    )
    node_mlp = _tc_call(
        _node_mlp_body, 3, N // BN,
        [
            _row_spec(BN), _row_spec(BN), _row_spec(BN),
            full((H, H)), full((H, H)), full((1, H)), full((H, H)), full((1, H)),
            full((1, H)), full((1, H)), full((H, H)), full((H, H)),
        ],
        BN,
    )

    lp = l0["edge_mlp"]
    ga, gb = _sc_gather(a_tbl, b_tbl, dst, src)
    (upd,) = edge_mlp(
        ga, gb, ea,
        lp["l0"]["w"][2 * H:], _rowvec(lp["l0"]["b"]),
        lp["l1"]["w"], _rowvec(lp["l1"]["b"]),
        _rowvec(lp["ln"]["g"]), _rowvec(lp["ln"]["b"]),
    )
    agg2 = _sc_scatter(upd, src, zeros_stripe)
    np0 = l0["node_mlp"]
    xe, a_tbl, b_tbl = node_mlp(
        xe, agg2[0], agg2[1],
        np0["l0"]["w"][:H], np0["l0"]["w"][H:], _rowvec(np0["l0"]["b"]),
        np0["l1"]["w"], _rowvec(np0["l1"]["b"]),
        _rowvec(np0["ln"]["g"]), _rowvec(np0["ln"]["b"]),
        w0e1[:H], w0e1[H:2 * H],
    )
    ea = upd

    lp = l1["edge_mlp"]
    ga, gb = _sc_gather(a_tbl, b_tbl, dst, src)
    (upd,) = edge_mlp(
        ga, gb, ea,
        lp["l0"]["w"][2 * H:], _rowvec(lp["l0"]["b"]),
        lp["l1"]["w"], _rowvec(lp["l1"]["b"]),
        _rowvec(lp["ln"]["g"]), _rowvec(lp["ln"]["b"]),
    )
    agg2 = _sc_scatter(upd, src, zeros_stripe)

    dw1 = jnp.zeros((H, H), jnp.float32).at[:, :DOUT].set(pd["l1"]["w"])
    db1 = jnp.zeros((H,), jnp.float32).at[:DOUT].set(pd["l1"]["b"])
    np1 = l1["node_mlp"]
    node_dec = _tc_call(
        _node_dec_body, 1, N // BN,
        [
            _row_spec(BN), _row_spec(BN), _row_spec(BN),
            full((H, H)), full((H, H)), full((1, H)), full((H, H)), full((1, H)),
            full((1, H)), full((1, H)),
            full((H, H)), full((1, H)), full((H, H)), full((1, H)),
        ],
        BN,
    )
    (out,) = node_dec(
        xe, agg2[0], agg2[1],
        np1["l0"]["w"][:H], np1["l0"]["w"][H:], _rowvec(np1["l0"]["b"]),
        np1["l1"]["w"], _rowvec(np1["l1"]["b"]),
        _rowvec(np1["ln"]["g"]), _rowvec(np1["ln"]["b"]),
        pd["l0"]["w"], _rowvec(pd["l0"]["b"]), dw1, _rowvec(db1),
    )
    return out[:, :DOUT]


# speedup vs baseline: 4.0507x; 1.1462x over previous
import functools

import jax
import jax.numpy as jnp
from jax import lax
from jax.experimental import pallas as pl
from jax.experimental.pallas import tpu as pltpu
from jax.experimental.pallas import tpu_sc as plsc

N = 10000
E = 320000
H = 128
DOUT = 3

BE = 2560
BN = 2000

NC = 2
NS = 16
NW = NC * NS
EPW = E // NW
C = 80
NCHUNK = EPW // C
NPAD = 10240
ROWS_PER_SUB = NPAD // NS


def _ln(u, g, b):
    m = jnp.mean(u, axis=-1, keepdims=True)
    v = jnp.mean((u - m) ** 2, axis=-1, keepdims=True)
    return (u - m) * lax.rsqrt(v + 1e-5) * g + b


def _f32(*shape):
    return jax.ShapeDtypeStruct(shape, jnp.float32)


def _node_enc_body(x, mu, sig, w0, b0, w1, b1, g, b, wa, wb, xe, a, bt):
    xn = (x[...] - mu[...]) / sig[...]
    h = jnp.maximum(jnp.dot(xn, w0[...], preferred_element_type=jnp.float32) + b0[...], 0.0)
    u = jnp.dot(h, w1[...], preferred_element_type=jnp.float32) + b1[...]
    o = _ln(u, g[...], b[...])
    xe[...] = o
    a[...] = jnp.dot(o, wa[...], preferred_element_type=jnp.float32)
    bt[...] = jnp.dot(o, wb[...], preferred_element_type=jnp.float32)


def _edge_enc_body(ea, w0, b0, w1, b1, g, b, out):
    h = jnp.maximum(jnp.dot(ea[...], w0[...], preferred_element_type=jnp.float32) + b0[...], 0.0)
    u = jnp.dot(h, w1[...], preferred_element_type=jnp.float32) + b1[...]
    out[...] = _ln(u, g[...], b[...])


def _edge_mlp_body(ga, gb, ea, w0, b0, w1, b1, g, b, out):
    s = ga[...] + gb[...] + jnp.dot(ea[...], w0[...], preferred_element_type=jnp.float32) + b0[...]
    h = jnp.maximum(s, 0.0)
    u = jnp.dot(h, w1[...], preferred_element_type=jnp.float32) + b1[...]
    out[...] = _ln(u, g[...], b[...]) + ea[...]


def _node_mlp_body(x, a0, a1, w0x, w0a, b0, w1, b1, g, b, wa, wb, xo, a, bt):
    agg = a0[...] + a1[...]
    s = (jnp.dot(x[...], w0x[...], preferred_element_type=jnp.float32)
         + jnp.dot(agg, w0a[...], preferred_element_type=jnp.float32) + b0[...])
    h = jnp.maximum(s, 0.0)
    u = jnp.dot(h, w1[...], preferred_element_type=jnp.float32) + b1[...]
    xn = x[...] + _ln(u, g[...], b[...])
    xo[...] = xn
    a[...] = jnp.dot(xn, wa[...], preferred_element_type=jnp.float32)
    bt[...] = jnp.dot(xn, wb[...], preferred_element_type=jnp.float32)


def _node_dec_body(x, a0, a1, w0x, w0a, b0, w1, b1, g, b, dw0, db0, dw1, db1, out):
    agg = a0[...] + a1[...]
    s = (jnp.dot(x[...], w0x[...], preferred_element_type=jnp.float32)
         + jnp.dot(agg, w0a[...], preferred_element_type=jnp.float32) + b0[...])
    h = jnp.maximum(s, 0.0)
    u = jnp.dot(h, w1[...], preferred_element_type=jnp.float32) + b1[...]
    xn = x[...] + _ln(u, g[...], b[...])
    dh = jnp.maximum(jnp.dot(xn, dw0[...], preferred_element_type=jnp.float32) + db0[...], 0.0)
    out[...] = jnp.dot(dh, dw1[...], preferred_element_type=jnp.float32) + db1[...]


def _row_spec(rows):
    return pl.BlockSpec((rows, H), lambda i: (i, 0))


def _full_spec(shape):
    nd = len(shape)
    return pl.BlockSpec(shape, lambda i: (0,) * nd)


def _tc_call(body, n_out, grid, in_specs, out_rows, interpret=False):
    return pl.pallas_call(
        body,
        grid=(grid,),
        in_specs=in_specs,
        out_specs=[_row_spec(out_rows)] * n_out,
        out_shape=[_f32(grid * out_rows, H)] * n_out,
        interpret=interpret,
    )


def _sc_gather(a_tbl, b_tbl, dst, src):
    mesh = plsc.VectorSubcoreMesh(core_axis_name="c", subcore_axis_name="s")

    @functools.partial(
        pl.kernel,
        out_type=[_f32(E, H), _f32(E, H)],
        mesh=mesh,
        scratch_types=[
            pltpu.VMEM((EPW,), jnp.int32),
            pltpu.VMEM((EPW,), jnp.int32),
            pltpu.VMEM((C, H), jnp.float32),
            pltpu.VMEM((C, H), jnp.float32),
            pltpu.VMEM((C, H), jnp.float32),
            pltpu.VMEM((C, H), jnp.float32),
            pltpu.SemaphoreType.DMA,
            pltpu.SemaphoreType.DMA,
            pltpu.SemaphoreType.DMA,
            pltpu.SemaphoreType.DMA,
            pltpu.SemaphoreType.DMA,
            pltpu.SemaphoreType.DMA,
            pltpu.SemaphoreType.DMA,
            pltpu.SemaphoreType.DMA,
        ],
    )
    def k(a_hbm, b_hbm, dst_hbm, src_hbm, ga_hbm, gb_hbm,
          didx, sidx, a0, b0, a1, b1,
          sga0, sgb0, sga1, sgb1, wa0, wb0, wa1, wb1):
        wid = lax.axis_index("s") * NC + lax.axis_index("c")
        base = wid * EPW
        pltpu.sync_copy(dst_hbm.at[pl.ds(base, EPW)], didx)
        pltpu.sync_copy(src_hbm.at[pl.ds(base, EPW)], sidx)

        def gather(c, buf_a, buf_b, sa, sb):
            pltpu.async_copy(a_hbm.at[didx.at[pl.ds(c * C, C)]], buf_a, sa)
            pltpu.async_copy(b_hbm.at[sidx.at[pl.ds(c * C, C)]], buf_b, sb)

        def write(c, buf_a, buf_b, sa, sb):
            pltpu.async_copy(buf_a, ga_hbm.at[pl.ds(base + c * C, C)], sa)
            pltpu.async_copy(buf_b, gb_hbm.at[pl.ds(base + c * C, C)], sb)

        def drain_write(buf_a, buf_b, sa, sb):
            pltpu.make_async_copy(buf_a, ga_hbm.at[pl.ds(base, C)], sa).wait()
            pltpu.make_async_copy(buf_b, gb_hbm.at[pl.ds(base, C)], sb).wait()

        def wait_gather(buf_a, buf_b, sa, sb):
            pltpu.make_async_copy(a_hbm.at[didx.at[pl.ds(0, C)]], buf_a, sa).wait()
            pltpu.make_async_copy(b_hbm.at[sidx.at[pl.ds(0, C)]], buf_b, sb).wait()

        gather(0, a0, b0, sga0, sgb0)

        def step(i, _):
            @pl.when(i > 0)
            def _():
                drain_write(a1, b1, wa1, wb1)
            gather(2 * i + 1, a1, b1, sga1, sgb1)
            wait_gather(a0, b0, sga0, sgb0)
            write(2 * i, a0, b0, wa0, wb0)
            drain_write(a0, b0, wa0, wb0)
            gather(2 * i + 2, a0, b0, sga0, sgb0)
            wait_gather(a1, b1, sga1, sgb1)
            write(2 * i + 1, a1, b1, wa1, wb1)
            return 0

        lax.fori_loop(0, (NCHUNK - 1) // 2, step, 0)
        drain_write(a1, b1, wa1, wb1)
        wait_gather(a0, b0, sga0, sgb0)
        write(NCHUNK - 1, a0, b0, wa0, wb0)
        drain_write(a0, b0, wa0, wb0)

    return k(a_tbl, b_tbl, dst, src)


def _sc_scatter(upd, src, zeros_stripe):
    mesh = plsc.VectorSubcoreMesh(core_axis_name="c", subcore_axis_name="s")

    @functools.partial(
        pl.kernel,
        out_type=_f32(NC, NPAD, H),
        mesh=mesh,
        scratch_types=[
            pltpu.VMEM((EPW,), jnp.int32),
            pltpu.VMEM((C, H), jnp.float32),
            pltpu.VMEM((C, H), jnp.float32),
            pltpu.SemaphoreType.DMA,
            pltpu.SemaphoreType.DMA,
            pltpu.VMEM_SHARED((NPAD, H), jnp.float32),
        ],
    )
    def k(upd_hbm, src_hbm, z_hbm, agg_hbm, sidx, buf0, buf1, r0, r1, shared):
        cid = lax.axis_index("c")
        sid = lax.axis_index("s")
        wid = sid * NC + cid
        base = wid * EPW
        stripe = sid * ROWS_PER_SUB
        pltpu.sync_copy(z_hbm, shared.at[pl.ds(stripe, ROWS_PER_SUB)])
        pltpu.sync_copy(src_hbm.at[pl.ds(base, EPW)], sidx)
        plsc.subcore_barrier()

        def read(c, buf, sem):
            pltpu.async_copy(upd_hbm.at[pl.ds(base + c * C, C)], buf, sem)

        def wait_read(buf, sem):
            pltpu.make_async_copy(upd_hbm.at[pl.ds(base, C)], buf, sem).wait()

        def scat(c, buf):
            pltpu.sync_copy(buf, shared.at[sidx.at[pl.ds(c * C, C)]], add=True)

        read(0, buf0, r0)

        def step(i, _):
            read(2 * i + 1, buf1, r1)
            wait_read(buf0, r0)
            scat(2 * i, buf0)
            read(2 * i + 2, buf0, r0)
            wait_read(buf1, r1)
            scat(2 * i + 1, buf1)
            return 0

        lax.fori_loop(0, (NCHUNK - 1) // 2, step, 0)
        wait_read(buf0, r0)
        scat(NCHUNK - 1, buf0)
        plsc.subcore_barrier()
        pltpu.sync_copy(shared.at[pl.ds(stripe, ROWS_PER_SUB)],
                        agg_hbm.at[cid, pl.ds(stripe, ROWS_PER_SUB)])

    return k(upd, src, zeros_stripe)


def _rowvec(v, width=H):
    return jnp.reshape(v, (1, width)).astype(jnp.float32)


def kernel(x, edge_index, edge_attr, mean_vec_x, std_vec_x, mean_vec_edge,
           std_vec_edge, params):
    dst = edge_index[1].astype(jnp.int32)
    src = edge_index[0].astype(jnp.int32)

    pe = params["node_enc"]
    pee = params["edge_enc"]
    l0 = params["layers"][0]
    l1 = params["layers"][1]
    pd = params["dec"]

    w0e0 = l0["edge_mlp"]["l0"]["w"]
    w0e1 = l1["edge_mlp"]["l0"]["w"]

    sig_e = std_vec_edge.astype(jnp.float32)
    w0_enc = pee["l0"]["w"] / sig_e[:, None]
    b0_enc = pee["l0"]["b"] - mean_vec_edge @ w0_enc
    w0_enc8 = jnp.zeros((8, H), jnp.float32).at[:4].set(w0_enc)
    ea8 = jnp.zeros((E, 8), jnp.float32).at[:, :4].set(edge_attr.astype(jnp.float32))

    full = _full_spec
    node_enc = _tc_call(
        _node_enc_body, 3, N // BN,
        [
            _row_spec(BN), full((1, H)), full((1, H)),
            full((H, H)), full((1, H)), full((H, H)), full((1, H)),
            full((1, H)), full((1, H)), full((H, H)), full((H, H)),
        ],
        BN,
    )
    xe, a_tbl, b_tbl = node_enc(
        x.astype(jnp.float32), _rowvec(mean_vec_x), _rowvec(std_vec_x),
        pe["l0"]["w"], _rowvec(pe["l0"]["b"]), pe["l1"]["w"], _rowvec(pe["l1"]["b"]),
        _rowvec(pe["ln"]["g"]), _rowvec(pe["ln"]["b"]),
        w0e0[:H], w0e0[H:2 * H],
    )

    edge_enc = _tc_call(
        _edge_enc_body, 1, E // BE,
        [
            pl.BlockSpec((BE, 8), lambda i: (i, 0)), pl.BlockSpec((8, H), lambda i: (0, 0)),
            full((1, H)), full((H, H)), full((1, H)), full((1, H)), full((1, H)),
        ],
        BE,
    )
    (ea,) = edge_enc(
        ea8, w0_enc8, _rowvec(b0_enc),
        pee["l1"]["w"], _rowvec(pee["l1"]["b"]),
        _rowvec(pee["ln"]["g"]), _rowvec(pee["ln"]["b"]),
    )

    zeros_stripe = jnp.zeros((ROWS_PER_SUB, H), jnp.float32)

    edge_mlp = _tc_call(
        _edge_mlp_body, 1, E // BE,
        [
            _row_spec(BE), _row_spec(BE), _row_spec(BE),
            full((H, H)), full((1, H)), full((H, H)), full((1, H)),
            full((1, H)), full((1, H)),
        ],
        BE,
    )
    node_mlp = _tc_call(
        _node_mlp_body, 3, N // BN,
        [
            _row_spec(BN), _row_spec(BN), _row_spec(BN),
            full((H, H)), full((H, H)), full((1, H)), full((H, H)), full((1, H)),
            full((1, H)), full((1, H)), full((H, H)), full((H, H)),
        ],
        BN,
    )

    lp = l0["edge_mlp"]
    ga, gb = _sc_gather(a_tbl, b_tbl, dst, src)
    (upd,) = edge_mlp(
        ga, gb, ea,
        lp["l0"]["w"][2 * H:], _rowvec(lp["l0"]["b"]),
        lp["l1"]["w"], _rowvec(lp["l1"]["b"]),
        _rowvec(lp["ln"]["g"]), _rowvec(lp["ln"]["b"]),
    )
    agg2 = _sc_scatter(upd, src, zeros_stripe)
    np0 = l0["node_mlp"]
    xe, a_tbl, b_tbl = node_mlp(
        xe, agg2[0], agg2[1],
        np0["l0"]["w"][:H], np0["l0"]["w"][H:], _rowvec(np0["l0"]["b"]),
        np0["l1"]["w"], _rowvec(np0["l1"]["b"]),
        _rowvec(np0["ln"]["g"]), _rowvec(np0["ln"]["b"]),
        w0e1[:H], w0e1[H:2 * H],
    )
    ea = upd

    lp = l1["edge_mlp"]
    ga, gb = _sc_gather(a_tbl, b_tbl, dst, src)
    (upd,) = edge_mlp(
        ga, gb, ea,
        lp["l0"]["w"][2 * H:], _rowvec(lp["l0"]["b"]),
        lp["l1"]["w"], _rowvec(lp["l1"]["b"]),
        _rowvec(lp["ln"]["g"]), _rowvec(lp["ln"]["b"]),
    )
    agg2 = _sc_scatter(upd, src, zeros_stripe)

    dw1 = jnp.zeros((H, H), jnp.float32).at[:, :DOUT].set(pd["l1"]["w"])
    db1 = jnp.zeros((H,), jnp.float32).at[:DOUT].set(pd["l1"]["b"])
    np1 = l1["node_mlp"]
    node_dec = _tc_call(
        _node_dec_body, 1, N // BN,
        [
            _row_spec(BN), _row_spec(BN), _row_spec(BN),
            full((H, H)), full((H, H)), full((1, H)), full((H, H)), full((1, H)),
            full((1, H)), full((1, H)),
            full((H, H)), full((1, H)), full((H, H)), full((1, H)),
        ],
        BN,
    )
    (out,) = node_dec(
        xe, agg2[0], agg2[1],
        np1["l0"]["w"][:H], np1["l0"]["w"][H:], _rowvec(np1["l0"]["b"]),
        np1["l1"]["w"], _rowvec(np1["l1"]["b"]),
        _rowvec(np1["ln"]["g"]), _rowvec(np1["ln"]["b"]),
        pd["l0"]["w"], _rowvec(pd["l0"]["b"]), dw1, _rowvec(db1),
    )
    return out[:, :DOUT]


# speedup vs baseline: 4.5637x; 1.1266x over previous
import functools

import jax
import jax.numpy as jnp
from jax import lax
from jax.experimental import pallas as pl
from jax.experimental.pallas import tpu as pltpu
from jax.experimental.pallas import tpu_sc as plsc

N = 10000
E = 320000
H = 128
DOUT = 3

BE = 2560
BN = 2000

NC = 2
NS = 16
NW = NC * NS
EPW = E // NW
C = 80
NCHUNK = EPW // C
NPAD = 10240
ROWS_PER_SUB = NPAD // NS


def _ln(u, g, b):
    m = jnp.mean(u, axis=-1, keepdims=True)
    v = jnp.mean((u - m) ** 2, axis=-1, keepdims=True)
    return (u - m) * lax.rsqrt(v + 1e-5) * g + b


def _f32(*shape):
    return jax.ShapeDtypeStruct(shape, jnp.float32)


def _node_enc_body(x, mu, sig, w0, b0, w1, b1, g, b, wa, wb, xe, a, bt):
    xn = (x[...] - mu[...]) / sig[...]
    h = jnp.maximum(jnp.dot(xn, w0[...], preferred_element_type=jnp.float32) + b0[...], 0.0)
    u = jnp.dot(h, w1[...], preferred_element_type=jnp.float32) + b1[...]
    o = _ln(u, g[...], b[...])
    xe[...] = o
    a[...] = jnp.dot(o, wa[...], preferred_element_type=jnp.float32)
    bt[...] = jnp.dot(o, wb[...], preferred_element_type=jnp.float32)


def _edge_enc_body(ea, w00, w01, w02, w03, b0, w1, b1, g, b, out):
    s = (b0[...] + ea[:, 0:1] * w00[...] + ea[:, 1:2] * w01[...]
         + ea[:, 2:3] * w02[...] + ea[:, 3:4] * w03[...])
    h = jnp.maximum(s, 0.0)
    u = jnp.dot(h, w1[...], preferred_element_type=jnp.float32) + b1[...]
    out[...] = _ln(u, g[...], b[...])


def _edge_mlp_body(ga, gb, ea, w0, b0, w1, b1, g, b, out):
    s = ga[...] + gb[...] + jnp.dot(ea[...], w0[...], preferred_element_type=jnp.float32) + b0[...]
    h = jnp.maximum(s, 0.0)
    u = jnp.dot(h, w1[...], preferred_element_type=jnp.float32) + b1[...]
    out[...] = _ln(u, g[...], b[...]) + ea[...]


def _node_mlp_body(x, a0, a1, w0x, w0a, b0, w1, b1, g, b, wa, wb, xo, a, bt):
    agg = a0[...] + a1[...]
    s = (jnp.dot(x[...], w0x[...], preferred_element_type=jnp.float32)
         + jnp.dot(agg, w0a[...], preferred_element_type=jnp.float32) + b0[...])
    h = jnp.maximum(s, 0.0)
    u = jnp.dot(h, w1[...], preferred_element_type=jnp.float32) + b1[...]
    xn = x[...] + _ln(u, g[...], b[...])
    xo[...] = xn
    a[...] = jnp.dot(xn, wa[...], preferred_element_type=jnp.float32)
    bt[...] = jnp.dot(xn, wb[...], preferred_element_type=jnp.float32)


def _node_dec_body(x, a0, a1, w0x, w0a, b0, w1, b1, g, b, dw0, db0, dw1, db1, out):
    agg = a0[...] + a1[...]
    s = (jnp.dot(x[...], w0x[...], preferred_element_type=jnp.float32)
         + jnp.dot(agg, w0a[...], preferred_element_type=jnp.float32) + b0[...])
    h = jnp.maximum(s, 0.0)
    u = jnp.dot(h, w1[...], preferred_element_type=jnp.float32) + b1[...]
    xn = x[...] + _ln(u, g[...], b[...])
    dh = jnp.maximum(jnp.dot(xn, dw0[...], preferred_element_type=jnp.float32) + db0[...], 0.0)
    out[...] = jnp.dot(dh, dw1[...], preferred_element_type=jnp.float32) + db1[...]


def _row_spec(rows):
    return pl.BlockSpec((rows, H), lambda i: (i, 0))


def _full_spec(shape):
    nd = len(shape)
    return pl.BlockSpec(shape, lambda i: (0,) * nd)


def _tc_call(body, n_out, grid, in_specs, out_rows, interpret=False):
    return pl.pallas_call(
        body,
        grid=(grid,),
        in_specs=in_specs,
        out_specs=[_row_spec(out_rows)] * n_out,
        out_shape=[_f32(grid * out_rows, H)] * n_out,
        interpret=interpret,
    )


def _sc_gather(a_tbl, b_tbl, dst, src):
    mesh = plsc.VectorSubcoreMesh(core_axis_name="c", subcore_axis_name="s")

    @functools.partial(
        pl.kernel,
        out_type=[_f32(E, H), _f32(E, H)],
        mesh=mesh,
        scratch_types=[
            pltpu.VMEM((EPW,), jnp.int32),
            pltpu.VMEM((EPW,), jnp.int32),
            pltpu.VMEM((C, H), jnp.float32),
            pltpu.VMEM((C, H), jnp.float32),
            pltpu.VMEM((C, H), jnp.float32),
            pltpu.VMEM((C, H), jnp.float32),
            pltpu.SemaphoreType.DMA,
            pltpu.SemaphoreType.DMA,
            pltpu.SemaphoreType.DMA,
            pltpu.SemaphoreType.DMA,
            pltpu.SemaphoreType.DMA,
            pltpu.SemaphoreType.DMA,
            pltpu.SemaphoreType.DMA,
            pltpu.SemaphoreType.DMA,
        ],
    )
    def k(a_hbm, b_hbm, dst_hbm, src_hbm, ga_hbm, gb_hbm,
          didx, sidx, a0, b0, a1, b1,
          sga0, sgb0, sga1, sgb1, wa0, wb0, wa1, wb1):
        wid = lax.axis_index("s") * NC + lax.axis_index("c")
        base = wid * EPW
        pltpu.sync_copy(dst_hbm.at[pl.ds(base, EPW)], didx)
        pltpu.sync_copy(src_hbm.at[pl.ds(base, EPW)], sidx)

        def gather(c, buf_a, buf_b, sa, sb):
            pltpu.async_copy(a_hbm.at[didx.at[pl.ds(c * C, C)]], buf_a, sa)
            pltpu.async_copy(b_hbm.at[sidx.at[pl.ds(c * C, C)]], buf_b, sb)

        def write(c, buf_a, buf_b, sa, sb):
            pltpu.async_copy(buf_a, ga_hbm.at[pl.ds(base + c * C, C)], sa)
            pltpu.async_copy(buf_b, gb_hbm.at[pl.ds(base + c * C, C)], sb)

        def drain_write(buf_a, buf_b, sa, sb):
            pltpu.make_async_copy(buf_a, ga_hbm.at[pl.ds(base, C)], sa).wait()
            pltpu.make_async_copy(buf_b, gb_hbm.at[pl.ds(base, C)], sb).wait()

        def wait_gather(buf_a, buf_b, sa, sb):
            pltpu.make_async_copy(a_hbm.at[didx.at[pl.ds(0, C)]], buf_a, sa).wait()
            pltpu.make_async_copy(b_hbm.at[sidx.at[pl.ds(0, C)]], buf_b, sb).wait()

        gather(0, a0, b0, sga0, sgb0)

        def step(i, _):
            @pl.when(i > 0)
            def _():
                drain_write(a1, b1, wa1, wb1)
            gather(2 * i + 1, a1, b1, sga1, sgb1)
            wait_gather(a0, b0, sga0, sgb0)
            write(2 * i, a0, b0, wa0, wb0)
            drain_write(a0, b0, wa0, wb0)
            gather(2 * i + 2, a0, b0, sga0, sgb0)
            wait_gather(a1, b1, sga1, sgb1)
            write(2 * i + 1, a1, b1, wa1, wb1)
            return 0

        lax.fori_loop(0, (NCHUNK - 1) // 2, step, 0)
        drain_write(a1, b1, wa1, wb1)
        wait_gather(a0, b0, sga0, sgb0)
        write(NCHUNK - 1, a0, b0, wa0, wb0)
        drain_write(a0, b0, wa0, wb0)

    return k(a_tbl, b_tbl, dst, src)


def _sc_scatter(upd, src, zeros_stripe):
    mesh = plsc.VectorSubcoreMesh(core_axis_name="c", subcore_axis_name="s")

    @functools.partial(
        pl.kernel,
        out_type=_f32(NC, NPAD, H),
        mesh=mesh,
        scratch_types=[
            pltpu.VMEM((EPW,), jnp.int32),
            pltpu.VMEM((C, H), jnp.float32),
            pltpu.VMEM((C, H), jnp.float32),
            pltpu.SemaphoreType.DMA,
            pltpu.SemaphoreType.DMA,
            pltpu.VMEM_SHARED((NPAD, H), jnp.float32),
        ],
    )
    def k(upd_hbm, src_hbm, z_hbm, agg_hbm, sidx, buf0, buf1, r0, r1, shared):
        cid = lax.axis_index("c")
        sid = lax.axis_index("s")
        wid = sid * NC + cid
        base = wid * EPW
        stripe = sid * ROWS_PER_SUB
        pltpu.sync_copy(z_hbm, shared.at[pl.ds(stripe, ROWS_PER_SUB)])
        pltpu.sync_copy(src_hbm.at[pl.ds(base, EPW)], sidx)
        plsc.subcore_barrier()

        def read(c, buf, sem):
            pltpu.async_copy(upd_hbm.at[pl.ds(base + c * C, C)], buf, sem)

        def wait_read(buf, sem):
            pltpu.make_async_copy(upd_hbm.at[pl.ds(base, C)], buf, sem).wait()

        def scat(c, buf):
            pltpu.sync_copy(buf, shared.at[sidx.at[pl.ds(c * C, C)]], add=True)

        read(0, buf0, r0)

        def step(i, _):
            read(2 * i + 1, buf1, r1)
            wait_read(buf0, r0)
            scat(2 * i, buf0)
            read(2 * i + 2, buf0, r0)
            wait_read(buf1, r1)
            scat(2 * i + 1, buf1)
            return 0

        lax.fori_loop(0, (NCHUNK - 1) // 2, step, 0)
        wait_read(buf0, r0)
        scat(NCHUNK - 1, buf0)
        plsc.subcore_barrier()
        pltpu.sync_copy(shared.at[pl.ds(stripe, ROWS_PER_SUB)],
                        agg_hbm.at[cid, pl.ds(stripe, ROWS_PER_SUB)])

    return k(upd, src, zeros_stripe)


def _rowvec(v, width=H):
    return jnp.reshape(v, (1, width)).astype(jnp.float32)


def kernel(x, edge_index, edge_attr, mean_vec_x, std_vec_x, mean_vec_edge,
           std_vec_edge, params):
    dst = edge_index[1].astype(jnp.int32)
    src = edge_index[0].astype(jnp.int32)

    pe = params["node_enc"]
    pee = params["edge_enc"]
    l0 = params["layers"][0]
    l1 = params["layers"][1]
    pd = params["dec"]

    w0e0 = l0["edge_mlp"]["l0"]["w"]
    w0e1 = l1["edge_mlp"]["l0"]["w"]

    sig_e = std_vec_edge.astype(jnp.float32)
    w0_enc = pee["l0"]["w"] / sig_e[:, None]
    b0_enc = pee["l0"]["b"] - mean_vec_edge @ w0_enc

    full = _full_spec
    node_enc = _tc_call(
        _node_enc_body, 3, N // BN,
        [
            _row_spec(BN), full((1, H)), full((1, H)),
            full((H, H)), full((1, H)), full((H, H)), full((1, H)),
            full((1, H)), full((1, H)), full((H, H)), full((H, H)),
        ],
        BN,
    )
    xe, a_tbl, b_tbl = node_enc(
        x.astype(jnp.float32), _rowvec(mean_vec_x), _rowvec(std_vec_x),
        pe["l0"]["w"], _rowvec(pe["l0"]["b"]), pe["l1"]["w"], _rowvec(pe["l1"]["b"]),
        _rowvec(pe["ln"]["g"]), _rowvec(pe["ln"]["b"]),
        w0e0[:H], w0e0[H:2 * H],
    )

    edge_enc = _tc_call(
        _edge_enc_body, 1, E // BE,
        [
            pl.BlockSpec((BE, 4), lambda i: (i, 0)),
            full((1, H)), full((1, H)), full((1, H)), full((1, H)),
            full((1, H)), full((H, H)), full((1, H)), full((1, H)), full((1, H)),
        ],
        BE,
    )
    (ea,) = edge_enc(
        edge_attr.astype(jnp.float32),
        _rowvec(w0_enc[0]), _rowvec(w0_enc[1]), _rowvec(w0_enc[2]), _rowvec(w0_enc[3]),
        _rowvec(b0_enc),
        pee["l1"]["w"], _rowvec(pee["l1"]["b"]),
        _rowvec(pee["ln"]["g"]), _rowvec(pee["ln"]["b"]),
    )

    zeros_stripe = jnp.zeros((ROWS_PER_SUB, H), jnp.float32)

    edge_mlp = _tc_call(
        _edge_mlp_body, 1, E // BE,
        [
            _row_spec(BE), _row_spec(BE), _row_spec(BE),
            full((H, H)), full((1, H)), full((H, H)), full((1, H)),
            full((1, H)), full((1, H)),
        ],
        BE,
    )
    node_mlp = _tc_call(
        _node_mlp_body, 3, N // BN,
        [
            _row_spec(BN), _row_spec(BN), _row_spec(BN),
            full((H, H)), full((H, H)), full((1, H)), full((H, H)), full((1, H)),
            full((1, H)), full((1, H)), full((H, H)), full((H, H)),
        ],
        BN,
    )

    lp = l0["edge_mlp"]
    ga, gb = _sc_gather(a_tbl, b_tbl, dst, src)
    (upd,) = edge_mlp(
        ga, gb, ea,
        lp["l0"]["w"][2 * H:], _rowvec(lp["l0"]["b"]),
        lp["l1"]["w"], _rowvec(lp["l1"]["b"]),
        _rowvec(lp["ln"]["g"]), _rowvec(lp["ln"]["b"]),
    )
    agg2 = _sc_scatter(upd, src, zeros_stripe)
    np0 = l0["node_mlp"]
    xe, a_tbl, b_tbl = node_mlp(
        xe, agg2[0], agg2[1],
        np0["l0"]["w"][:H], np0["l0"]["w"][H:], _rowvec(np0["l0"]["b"]),
        np0["l1"]["w"], _rowvec(np0["l1"]["b"]),
        _rowvec(np0["ln"]["g"]), _rowvec(np0["ln"]["b"]),
        w0e1[:H], w0e1[H:2 * H],
    )
    ea = upd

    lp = l1["edge_mlp"]
    ga, gb = _sc_gather(a_tbl, b_tbl, dst, src)
    (upd,) = edge_mlp(
        ga, gb, ea,
        lp["l0"]["w"][2 * H:], _rowvec(lp["l0"]["b"]),
        lp["l1"]["w"], _rowvec(lp["l1"]["b"]),
        _rowvec(lp["ln"]["g"]), _rowvec(lp["ln"]["b"]),
    )
    agg2 = _sc_scatter(upd, src, zeros_stripe)

    dw1 = jnp.zeros((H, H), jnp.float32).at[:, :DOUT].set(pd["l1"]["w"])
    db1 = jnp.zeros((H,), jnp.float32).at[:DOUT].set(pd["l1"]["b"])
    np1 = l1["node_mlp"]
    node_dec = _tc_call(
        _node_dec_body, 1, N // BN,
        [
            _row_spec(BN), _row_spec(BN), _row_spec(BN),
            full((H, H)), full((H, H)), full((1, H)), full((H, H)), full((1, H)),
            full((1, H)), full((1, H)),
            full((H, H)), full((1, H)), full((H, H)), full((1, H)),
        ],
        BN,
    )
    (out,) = node_dec(
        xe, agg2[0], agg2[1],
        np1["l0"]["w"][:H], np1["l0"]["w"][H:], _rowvec(np1["l0"]["b"]),
        np1["l1"]["w"], _rowvec(np1["l1"]["b"]),
        _rowvec(np1["ln"]["g"]), _rowvec(np1["ln"]["b"]),
        pd["l0"]["w"], _rowvec(pd["l0"]["b"]), dw1, _rowvec(db1),
    )
    return out[:, :DOUT]


# speedup vs baseline: 4.6782x; 1.0251x over previous
import functools

import jax
import jax.numpy as jnp
from jax import lax
from jax.experimental import pallas as pl
from jax.experimental.pallas import tpu as pltpu
from jax.experimental.pallas import tpu_sc as plsc

N = 10000
E = 320000
H = 128
DOUT = 3

BE = 2560
BN = 2000

NC = 2
NS = 16
NW = NC * NS
C = 80
NPAD = 10240
ROWS_PER_SUB = NPAD // NS

CHUNK_PIECES = [41, 41, 43]
CHUNK_SIZES = [p * BE for p in CHUNK_PIECES]


def _ln(u, g, b):
    m = jnp.mean(u, axis=-1, keepdims=True)
    v = jnp.mean((u - m) ** 2, axis=-1, keepdims=True)
    return (u - m) * lax.rsqrt(v + 1e-5) * g + b


def _f32(*shape):
    return jax.ShapeDtypeStruct(shape, jnp.float32)


NAGG = 2 * len(CHUNK_SIZES)


def _node_enc_body(x, mu, sig, w0, b0, w1, b1, g, b, wa, wb, xe, a, bt):
    xn = (x[...] - mu[...]) / sig[...]
    h = jnp.maximum(jnp.dot(xn, w0[...], preferred_element_type=jnp.float32) + b0[...], 0.0)
    u = jnp.dot(h, w1[...], preferred_element_type=jnp.float32) + b1[...]
    o = _ln(u, g[...], b[...])
    xe[...] = o
    a[...] = jnp.dot(o, wa[...], preferred_element_type=jnp.float32)
    bt[...] = jnp.dot(o, wb[...], preferred_element_type=jnp.float32)


def _edge_enc_body(ea, w00, w01, w02, w03, b0, w1, b1, g, b, out):
    s = (b0[...] + ea[:, 0:1] * w00[...] + ea[:, 1:2] * w01[...]
         + ea[:, 2:3] * w02[...] + ea[:, 3:4] * w03[...])
    h = jnp.maximum(s, 0.0)
    u = jnp.dot(h, w1[...], preferred_element_type=jnp.float32) + b1[...]
    out[...] = _ln(u, g[...], b[...])


def _edge_mlp_body(ga, gb, ea, w0, b0, w1, b1, g, b, out):
    s = ga[...] + gb[...] + jnp.dot(ea[...], w0[...], preferred_element_type=jnp.float32) + b0[...]
    h = jnp.maximum(s, 0.0)
    u = jnp.dot(h, w1[...], preferred_element_type=jnp.float32) + b1[...]
    out[...] = _ln(u, g[...], b[...]) + ea[...]


def _node_mlp_body(*refs):
    x = refs[0]
    aggs = refs[1:1 + NAGG]
    w0x, w0a, b0, w1, b1, g, b, wa, wb = refs[1 + NAGG:10 + NAGG]
    xo, a, bt = refs[10 + NAGG:]
    agg = aggs[0][...]
    for r in aggs[1:]:
        agg = agg + r[...]
    s = (jnp.dot(x[...], w0x[...], preferred_element_type=jnp.float32)
         + jnp.dot(agg, w0a[...], preferred_element_type=jnp.float32) + b0[...])
    h = jnp.maximum(s, 0.0)
    u = jnp.dot(h, w1[...], preferred_element_type=jnp.float32) + b1[...]
    xn = x[...] + _ln(u, g[...], b[...])
    xo[...] = xn
    a[...] = jnp.dot(xn, wa[...], preferred_element_type=jnp.float32)
    bt[...] = jnp.dot(xn, wb[...], preferred_element_type=jnp.float32)


def _node_dec_body(*refs):
    x = refs[0]
    aggs = refs[1:1 + NAGG]
    w0x, w0a, b0, w1, b1, g, b, dw0, db0, dw1, db1 = refs[1 + NAGG:12 + NAGG]
    out = refs[12 + NAGG]
    agg = aggs[0][...]
    for r in aggs[1:]:
        agg = agg + r[...]
    s = (jnp.dot(x[...], w0x[...], preferred_element_type=jnp.float32)
         + jnp.dot(agg, w0a[...], preferred_element_type=jnp.float32) + b0[...])
    h = jnp.maximum(s, 0.0)
    u = jnp.dot(h, w1[...], preferred_element_type=jnp.float32) + b1[...]
    xn = x[...] + _ln(u, g[...], b[...])
    dh = jnp.maximum(jnp.dot(xn, dw0[...], preferred_element_type=jnp.float32) + db0[...], 0.0)
    out[...] = jnp.dot(dh, dw1[...], preferred_element_type=jnp.float32) + db1[...]


def _row_spec(rows, off_blocks=0):
    if off_blocks:
        return pl.BlockSpec((rows, H), lambda i: (i + off_blocks, 0))
    return pl.BlockSpec((rows, H), lambda i: (i, 0))


def _full_spec(shape):
    nd = len(shape)
    return pl.BlockSpec(shape, lambda i: (0,) * nd)


def _tc_call(body, n_out, grid, in_specs, out_rows, interpret=False):
    return pl.pallas_call(
        body,
        grid=(grid,),
        in_specs=in_specs,
        out_specs=[_row_spec(out_rows)] * n_out,
        out_shape=[_f32(grid * out_rows, H)] * n_out,
        interpret=interpret,
    )


def _sc_gather(a_tbl, b_tbl, dst, src):
    ec = dst.shape[0]
    epw = ec // NW
    nchunk = epw // C
    assert nchunk % 2 == 1 and nchunk * C == epw
    mesh = plsc.VectorSubcoreMesh(core_axis_name="c", subcore_axis_name="s")

    @functools.partial(
        pl.kernel,
        out_type=[_f32(ec, H), _f32(ec, H)],
        mesh=mesh,
        scratch_types=[
            pltpu.VMEM((epw,), jnp.int32),
            pltpu.VMEM((epw,), jnp.int32),
            pltpu.VMEM((C, H), jnp.float32),
            pltpu.VMEM((C, H), jnp.float32),
            pltpu.VMEM((C, H), jnp.float32),
            pltpu.VMEM((C, H), jnp.float32),
            pltpu.SemaphoreType.DMA,
            pltpu.SemaphoreType.DMA,
            pltpu.SemaphoreType.DMA,
            pltpu.SemaphoreType.DMA,
            pltpu.SemaphoreType.DMA,
            pltpu.SemaphoreType.DMA,
            pltpu.SemaphoreType.DMA,
            pltpu.SemaphoreType.DMA,
        ],
    )
    def k(a_hbm, b_hbm, dst_hbm, src_hbm, ga_hbm, gb_hbm,
          didx, sidx, a0, b0, a1, b1,
          sga0, sgb0, sga1, sgb1, wa0, wb0, wa1, wb1):
        wid = lax.axis_index("s") * NC + lax.axis_index("c")
        base = wid * epw
        pltpu.sync_copy(dst_hbm.at[pl.ds(base, epw)], didx)
        pltpu.sync_copy(src_hbm.at[pl.ds(base, epw)], sidx)

        def gather(c, buf_a, buf_b, sa, sb):
            pltpu.async_copy(a_hbm.at[didx.at[pl.ds(c * C, C)]], buf_a, sa)
            pltpu.async_copy(b_hbm.at[sidx.at[pl.ds(c * C, C)]], buf_b, sb)

        def write(c, buf_a, buf_b, sa, sb):
            pltpu.async_copy(buf_a, ga_hbm.at[pl.ds(base + c * C, C)], sa)
            pltpu.async_copy(buf_b, gb_hbm.at[pl.ds(base + c * C, C)], sb)

        def drain_write(buf_a, buf_b, sa, sb):
            pltpu.make_async_copy(buf_a, ga_hbm.at[pl.ds(base, C)], sa).wait()
            pltpu.make_async_copy(buf_b, gb_hbm.at[pl.ds(base, C)], sb).wait()

        def wait_gather(buf_a, buf_b, sa, sb):
            pltpu.make_async_copy(a_hbm.at[didx.at[pl.ds(0, C)]], buf_a, sa).wait()
            pltpu.make_async_copy(b_hbm.at[sidx.at[pl.ds(0, C)]], buf_b, sb).wait()

        gather(0, a0, b0, sga0, sgb0)

        def step(i, _):
            @pl.when(i > 0)
            def _():
                drain_write(a1, b1, wa1, wb1)
            gather(2 * i + 1, a1, b1, sga1, sgb1)
            wait_gather(a0, b0, sga0, sgb0)
            write(2 * i, a0, b0, wa0, wb0)
            drain_write(a0, b0, wa0, wb0)
            gather(2 * i + 2, a0, b0, sga0, sgb0)
            wait_gather(a1, b1, sga1, sgb1)
            write(2 * i + 1, a1, b1, wa1, wb1)
            return 0

        lax.fori_loop(0, (nchunk - 1) // 2, step, 0)
        drain_write(a1, b1, wa1, wb1)
        wait_gather(a0, b0, sga0, sgb0)
        write(nchunk - 1, a0, b0, wa0, wb0)
        drain_write(a0, b0, wa0, wb0)

    return k(a_tbl, b_tbl, dst, src)


def _sc_scatter(upd, src, zeros_stripe):
    ec = src.shape[0]
    epw = ec // NW
    nchunk = epw // C
    assert nchunk % 2 == 1 and nchunk * C == epw
    mesh = plsc.VectorSubcoreMesh(core_axis_name="c", subcore_axis_name="s")

    @functools.partial(
        pl.kernel,
        out_type=_f32(NC, NPAD, H),
        mesh=mesh,
        scratch_types=[
            pltpu.VMEM((epw,), jnp.int32),
            pltpu.VMEM((C, H), jnp.float32),
            pltpu.VMEM((C, H), jnp.float32),
            pltpu.SemaphoreType.DMA,
            pltpu.SemaphoreType.DMA,
            pltpu.VMEM_SHARED((NPAD, H), jnp.float32),
        ],
    )
    def k(upd_hbm, src_hbm, z_hbm, agg_hbm, sidx, buf0, buf1, r0, r1, shared):
        cid = lax.axis_index("c")
        sid = lax.axis_index("s")
        wid = sid * NC + cid
        base = wid * epw
        stripe = sid * ROWS_PER_SUB
        pltpu.sync_copy(z_hbm, shared.at[pl.ds(stripe, ROWS_PER_SUB)])
        pltpu.sync_copy(src_hbm.at[pl.ds(base, epw)], sidx)
        plsc.subcore_barrier()

        def read(c, buf, sem):
            pltpu.async_copy(upd_hbm.at[pl.ds(base + c * C, C)], buf, sem)

        def wait_read(buf, sem):
            pltpu.make_async_copy(upd_hbm.at[pl.ds(base, C)], buf, sem).wait()

        def scat(c, buf):
            pltpu.sync_copy(buf, shared.at[sidx.at[pl.ds(c * C, C)]], add=True)

        read(0, buf0, r0)

        def step(i, _):
            read(2 * i + 1, buf1, r1)
            wait_read(buf0, r0)
            scat(2 * i, buf0)
            read(2 * i + 2, buf0, r0)
            wait_read(buf1, r1)
            scat(2 * i + 1, buf1)
            return 0

        lax.fori_loop(0, (nchunk - 1) // 2, step, 0)
        wait_read(buf0, r0)
        scat(nchunk - 1, buf0)
        plsc.subcore_barrier()
        pltpu.sync_copy(shared.at[pl.ds(stripe, ROWS_PER_SUB)],
                        agg_hbm.at[cid, pl.ds(stripe, ROWS_PER_SUB)])

    return k(upd, src, zeros_stripe)


def _rowvec(v, width=H):
    return jnp.reshape(v, (1, width)).astype(jnp.float32)


def _edge_layer(lp, a_tbl, b_tbl, ea_full, ea_chunks, dst_chunks, src_chunks,
                zeros_stripe):
    full = _full_spec
    upd_chunks = []
    aggs = []
    w0 = lp["l0"]["w"]
    off = 0
    for ci, ec in enumerate(CHUNK_SIZES):
        ga, gb = _sc_gather(a_tbl, b_tbl, dst_chunks[ci], src_chunks[ci])
        if ea_full is not None:
            ea_spec = _row_spec(BE, off // BE)
            ea_arg = ea_full
        else:
            ea_spec = _row_spec(BE)
            ea_arg = ea_chunks[ci]
        edge_mlp = _tc_call(
            _edge_mlp_body, 1, ec // BE,
            [
                _row_spec(BE), _row_spec(BE), ea_spec,
                full((H, H)), full((1, H)), full((H, H)), full((1, H)),
                full((1, H)), full((1, H)),
            ],
            BE,
        )
        (upd,) = edge_mlp(
            ga, gb, ea_arg,
            w0[2 * H:], _rowvec(lp["l0"]["b"]),
            lp["l1"]["w"], _rowvec(lp["l1"]["b"]),
            _rowvec(lp["ln"]["g"]), _rowvec(lp["ln"]["b"]),
        )
        upd_chunks.append(upd)
        agg2 = _sc_scatter(upd, src_chunks[ci], zeros_stripe)
        aggs.append(agg2[0])
        aggs.append(agg2[1])
        off += ec
    return upd_chunks, aggs


def kernel(x, edge_index, edge_attr, mean_vec_x, std_vec_x, mean_vec_edge,
           std_vec_edge, params):
    dst = edge_index[1].astype(jnp.int32)
    src = edge_index[0].astype(jnp.int32)
    dst_chunks, src_chunks = [], []
    off = 0
    for ec in CHUNK_SIZES:
        dst_chunks.append(lax.slice(dst, (off,), (off + ec,)))
        src_chunks.append(lax.slice(src, (off,), (off + ec,)))
        off += ec

    pe = params["node_enc"]
    pee = params["edge_enc"]
    l0 = params["layers"][0]
    l1 = params["layers"][1]
    pd = params["dec"]

    w0e0 = l0["edge_mlp"]["l0"]["w"]
    w0e1 = l1["edge_mlp"]["l0"]["w"]

    sig_e = std_vec_edge.astype(jnp.float32)
    w0_enc = pee["l0"]["w"] / sig_e[:, None]
    b0_enc = pee["l0"]["b"] - mean_vec_edge @ w0_enc

    full = _full_spec
    node_enc = _tc_call(
        _node_enc_body, 3, N // BN,
        [
            _row_spec(BN), full((1, H)), full((1, H)),
            full((H, H)), full((1, H)), full((H, H)), full((1, H)),
            full((1, H)), full((1, H)), full((H, H)), full((H, H)),
        ],
        BN,
    )
    xe, a_tbl, b_tbl = node_enc(
        x.astype(jnp.float32), _rowvec(mean_vec_x), _rowvec(std_vec_x),
        pe["l0"]["w"], _rowvec(pe["l0"]["b"]), pe["l1"]["w"], _rowvec(pe["l1"]["b"]),
        _rowvec(pe["ln"]["g"]), _rowvec(pe["ln"]["b"]),
        w0e0[:H], w0e0[H:2 * H],
    )

    edge_enc = _tc_call(
        _edge_enc_body, 1, E // BE,
        [
            pl.BlockSpec((BE, 4), lambda i: (i, 0)),
            full((1, H)), full((1, H)), full((1, H)), full((1, H)),
            full((1, H)), full((H, H)), full((1, H)), full((1, H)), full((1, H)),
        ],
        BE,
    )
    (ea,) = edge_enc(
        edge_attr.astype(jnp.float32),
        _rowvec(w0_enc[0]), _rowvec(w0_enc[1]), _rowvec(w0_enc[2]), _rowvec(w0_enc[3]),
        _rowvec(b0_enc),
        pee["l1"]["w"], _rowvec(pee["l1"]["b"]),
        _rowvec(pee["ln"]["g"]), _rowvec(pee["ln"]["b"]),
    )

    zeros_stripe = jnp.zeros((ROWS_PER_SUB, H), jnp.float32)

    node_mlp = _tc_call(
        _node_mlp_body, 3, N // BN,
        [_row_spec(BN)] + [_row_spec(BN)] * NAGG + [
            full((H, H)), full((H, H)), full((1, H)), full((H, H)), full((1, H)),
            full((1, H)), full((1, H)), full((H, H)), full((H, H)),
        ],
        BN,
    )

    upd_chunks, aggs = _edge_layer(
        l0["edge_mlp"], a_tbl, b_tbl, ea, None, dst_chunks, src_chunks, zeros_stripe)
    np0 = l0["node_mlp"]
    xe, a_tbl, b_tbl = node_mlp(
        xe, *aggs,
        np0["l0"]["w"][:H], np0["l0"]["w"][H:], _rowvec(np0["l0"]["b"]),
        np0["l1"]["w"], _rowvec(np0["l1"]["b"]),
        _rowvec(np0["ln"]["g"]), _rowvec(np0["ln"]["b"]),
        w0e1[:H], w0e1[H:2 * H],
    )

    upd_chunks, aggs = _edge_layer(
        l1["edge_mlp"], a_tbl, b_tbl, None, upd_chunks, dst_chunks, src_chunks,
        zeros_stripe)

    dw1 = jnp.zeros((H, H), jnp.float32).at[:, :DOUT].set(pd["l1"]["w"])
    db1 = jnp.zeros((H,), jnp.float32).at[:DOUT].set(pd["l1"]["b"])
    np1 = l1["node_mlp"]
    node_dec = _tc_call(
        _node_dec_body, 1, N // BN,
        [_row_spec(BN)] + [_row_spec(BN)] * NAGG + [
            full((H, H)), full((H, H)), full((1, H)), full((H, H)), full((1, H)),
            full((1, H)), full((1, H)),
            full((H, H)), full((1, H)), full((H, H)), full((1, H)),
        ],
        BN,
    )
    (out,) = node_dec(
        xe, *aggs,
        np1["l0"]["w"][:H], np1["l0"]["w"][H:], _rowvec(np1["l0"]["b"]),
        np1["l1"]["w"], _rowvec(np1["l1"]["b"]),
        _rowvec(np1["ln"]["g"]), _rowvec(np1["ln"]["b"]),
        pd["l0"]["w"], _rowvec(pd["l0"]["b"]), dw1, _rowvec(db1),
    )
    return out[:, :DOUT]


# speedup vs baseline: 4.9187x; 1.0514x over previous
import functools

import jax
import jax.numpy as jnp
from jax import lax
from jax.experimental import pallas as pl
from jax.experimental.pallas import tpu as pltpu
from jax.experimental.pallas import tpu_sc as plsc

N = 10000
E = 320000
H = 128
DOUT = 3

BE = 2560
BN = 2000

NC = 2
NS = 16
NW = NC * NS
C = 80
NPAD = 10240
ROWS_PER_SUB = NPAD // NS

CHUNK_PIECES = [41, 41, 43]
CHUNK_SIZES = [p * BE for p in CHUNK_PIECES]


def _ln(u, g, b):
    m = jnp.mean(u, axis=-1, keepdims=True)
    v = jnp.mean((u - m) ** 2, axis=-1, keepdims=True)
    return (u - m) * lax.rsqrt(v + 1e-5) * g + b


def _f32(*shape):
    return jax.ShapeDtypeStruct(shape, jnp.float32)


NAGG = 2 * len(CHUNK_SIZES)


def _node_enc_body(x, mu, sig, w0, b0, w1, b1, g, b, wa, wb, xe, a, bt):
    xn = (x[...] - mu[...]) / sig[...]
    h = jnp.maximum(jnp.dot(xn, w0[...], preferred_element_type=jnp.float32) + b0[...], 0.0)
    u = jnp.dot(h, w1[...], preferred_element_type=jnp.float32) + b1[...]
    o = _ln(u, g[...], b[...])
    xe[...] = o
    a[...] = jnp.dot(o, wa[...], preferred_element_type=jnp.float32)
    bt[...] = jnp.dot(o, wb[...], preferred_element_type=jnp.float32)


def _edge_enc_body(s, w1, b1, g, b, out):
    h = jnp.maximum(s[...], 0.0)
    u = jnp.dot(h, w1[...], preferred_element_type=jnp.float32) + b1[...]
    out[...] = _ln(u, g[...], b[...])


def _edge_mlp_body(ga, gb, ea, w0, b0, w1, b1, g, b, out):
    s = ga[...] + gb[...] + jnp.dot(ea[...], w0[...], preferred_element_type=jnp.float32) + b0[...]
    h = jnp.maximum(s, 0.0)
    u = jnp.dot(h, w1[...], preferred_element_type=jnp.float32) + b1[...]
    out[...] = _ln(u, g[...], b[...]) + ea[...]


def _node_mlp_body(*refs):
    x = refs[0]
    aggs = refs[1:1 + NAGG]
    w0x, w0a, b0, w1, b1, g, b, wa, wb = refs[1 + NAGG:10 + NAGG]
    xo, a, bt = refs[10 + NAGG:]
    agg = aggs[0][...]
    for r in aggs[1:]:
        agg = agg + r[...]
    s = (jnp.dot(x[...], w0x[...], preferred_element_type=jnp.float32)
         + jnp.dot(agg, w0a[...], preferred_element_type=jnp.float32) + b0[...])
    h = jnp.maximum(s, 0.0)
    u = jnp.dot(h, w1[...], preferred_element_type=jnp.float32) + b1[...]
    xn = x[...] + _ln(u, g[...], b[...])
    xo[...] = xn
    a[...] = jnp.dot(xn, wa[...], preferred_element_type=jnp.float32)
    bt[...] = jnp.dot(xn, wb[...], preferred_element_type=jnp.float32)


def _node_dec_body(*refs):
    x = refs[0]
    aggs = refs[1:1 + NAGG]
    w0x, w0a, b0, w1, b1, g, b, dw0, db0, dw1, db1 = refs[1 + NAGG:12 + NAGG]
    out = refs[12 + NAGG]
    agg = aggs[0][...]
    for r in aggs[1:]:
        agg = agg + r[...]
    s = (jnp.dot(x[...], w0x[...], preferred_element_type=jnp.float32)
         + jnp.dot(agg, w0a[...], preferred_element_type=jnp.float32) + b0[...])
    h = jnp.maximum(s, 0.0)
    u = jnp.dot(h, w1[...], preferred_element_type=jnp.float32) + b1[...]
    xn = x[...] + _ln(u, g[...], b[...])
    dh = jnp.maximum(jnp.dot(xn, dw0[...], preferred_element_type=jnp.float32) + db0[...], 0.0)
    out[...] = jnp.dot(dh, dw1[...], preferred_element_type=jnp.float32) + db1[...]


def _row_spec(rows, off_blocks=0):
    if off_blocks:
        return pl.BlockSpec((rows, H), lambda i: (i + off_blocks, 0))
    return pl.BlockSpec((rows, H), lambda i: (i, 0))


def _full_spec(shape):
    nd = len(shape)
    return pl.BlockSpec(shape, lambda i: (0,) * nd)


def _tc_call(body, n_out, grid, in_specs, out_rows, interpret=False):
    return pl.pallas_call(
        body,
        grid=(grid,),
        in_specs=in_specs,
        out_specs=[_row_spec(out_rows)] * n_out,
        out_shape=[_f32(grid * out_rows, H)] * n_out,
        interpret=interpret,
    )


def _sc_gather(a_tbl, b_tbl, dst, src):
    ec = dst.shape[0]
    epw = ec // NW
    nchunk = epw // C
    assert nchunk % 2 == 1 and nchunk * C == epw
    mesh = plsc.VectorSubcoreMesh(core_axis_name="c", subcore_axis_name="s")

    @functools.partial(
        pl.kernel,
        out_type=[_f32(ec, H), _f32(ec, H)],
        mesh=mesh,
        scratch_types=[
            pltpu.VMEM((epw,), jnp.int32),
            pltpu.VMEM((epw,), jnp.int32),
            pltpu.VMEM((C, H), jnp.float32),
            pltpu.VMEM((C, H), jnp.float32),
            pltpu.VMEM((C, H), jnp.float32),
            pltpu.VMEM((C, H), jnp.float32),
            pltpu.SemaphoreType.DMA,
            pltpu.SemaphoreType.DMA,
            pltpu.SemaphoreType.DMA,
            pltpu.SemaphoreType.DMA,
            pltpu.SemaphoreType.DMA,
            pltpu.SemaphoreType.DMA,
            pltpu.SemaphoreType.DMA,
            pltpu.SemaphoreType.DMA,
        ],
    )
    def k(a_hbm, b_hbm, dst_hbm, src_hbm, ga_hbm, gb_hbm,
          didx, sidx, a0, b0, a1, b1,
          sga0, sgb0, sga1, sgb1, wa0, wb0, wa1, wb1):
        wid = lax.axis_index("s") * NC + lax.axis_index("c")
        base = wid * epw
        pltpu.sync_copy(dst_hbm.at[pl.ds(base, epw)], didx)
        pltpu.sync_copy(src_hbm.at[pl.ds(base, epw)], sidx)

        def gather(c, buf_a, buf_b, sa, sb):
            pltpu.async_copy(a_hbm.at[didx.at[pl.ds(c * C, C)]], buf_a, sa)
            pltpu.async_copy(b_hbm.at[sidx.at[pl.ds(c * C, C)]], buf_b, sb)

        def write(c, buf_a, buf_b, sa, sb):
            pltpu.async_copy(buf_a, ga_hbm.at[pl.ds(base + c * C, C)], sa)
            pltpu.async_copy(buf_b, gb_hbm.at[pl.ds(base + c * C, C)], sb)

        def drain_write(buf_a, buf_b, sa, sb):
            pltpu.make_async_copy(buf_a, ga_hbm.at[pl.ds(base, C)], sa).wait()
            pltpu.make_async_copy(buf_b, gb_hbm.at[pl.ds(base, C)], sb).wait()

        def wait_gather(buf_a, buf_b, sa, sb):
            pltpu.make_async_copy(a_hbm.at[didx.at[pl.ds(0, C)]], buf_a, sa).wait()
            pltpu.make_async_copy(b_hbm.at[sidx.at[pl.ds(0, C)]], buf_b, sb).wait()

        gather(0, a0, b0, sga0, sgb0)

        def step(i, _):
            @pl.when(i > 0)
            def _():
                drain_write(a1, b1, wa1, wb1)
            gather(2 * i + 1, a1, b1, sga1, sgb1)
            wait_gather(a0, b0, sga0, sgb0)
            write(2 * i, a0, b0, wa0, wb0)
            drain_write(a0, b0, wa0, wb0)
            gather(2 * i + 2, a0, b0, sga0, sgb0)
            wait_gather(a1, b1, sga1, sgb1)
            write(2 * i + 1, a1, b1, wa1, wb1)
            return 0

        lax.fori_loop(0, (nchunk - 1) // 2, step, 0)
        drain_write(a1, b1, wa1, wb1)
        wait_gather(a0, b0, sga0, sgb0)
        write(nchunk - 1, a0, b0, wa0, wb0)
        drain_write(a0, b0, wa0, wb0)

    return k(a_tbl, b_tbl, dst, src)


def _sc_scatter(upd, src, zeros_stripe):
    ec = src.shape[0]
    epw = ec // NW
    nchunk = epw // C
    assert nchunk % 2 == 1 and nchunk * C == epw
    mesh = plsc.VectorSubcoreMesh(core_axis_name="c", subcore_axis_name="s")

    @functools.partial(
        pl.kernel,
        out_type=_f32(NC, NPAD, H),
        mesh=mesh,
        scratch_types=[
            pltpu.VMEM((epw,), jnp.int32),
            pltpu.VMEM((C, H), jnp.float32),
            pltpu.VMEM((C, H), jnp.float32),
            pltpu.SemaphoreType.DMA,
            pltpu.SemaphoreType.DMA,
            pltpu.VMEM_SHARED((NPAD, H), jnp.float32),
        ],
    )
    def k(upd_hbm, src_hbm, z_hbm, agg_hbm, sidx, buf0, buf1, r0, r1, shared):
        cid = lax.axis_index("c")
        sid = lax.axis_index("s")
        wid = sid * NC + cid
        base = wid * epw
        stripe = sid * ROWS_PER_SUB
        pltpu.sync_copy(z_hbm, shared.at[pl.ds(stripe, ROWS_PER_SUB)])
        pltpu.sync_copy(src_hbm.at[pl.ds(base, epw)], sidx)
        plsc.subcore_barrier()

        def read(c, buf, sem):
            pltpu.async_copy(upd_hbm.at[pl.ds(base + c * C, C)], buf, sem)

        def wait_read(buf, sem):
            pltpu.make_async_copy(upd_hbm.at[pl.ds(base, C)], buf, sem).wait()

        def scat(c, buf):
            pltpu.sync_copy(buf, shared.at[sidx.at[pl.ds(c * C, C)]], add=True)

        read(0, buf0, r0)

        def step(i, _):
            read(2 * i + 1, buf1, r1)
            wait_read(buf0, r0)
            scat(2 * i, buf0)
            read(2 * i + 2, buf0, r0)
            wait_read(buf1, r1)
            scat(2 * i + 1, buf1)
            return 0

        lax.fori_loop(0, (nchunk - 1) // 2, step, 0)
        wait_read(buf0, r0)
        scat(nchunk - 1, buf0)
        plsc.subcore_barrier()
        pltpu.sync_copy(shared.at[pl.ds(stripe, ROWS_PER_SUB)],
                        agg_hbm.at[cid, pl.ds(stripe, ROWS_PER_SUB)])

    return k(upd, src, zeros_stripe)


def _rowvec(v, width=H):
    return jnp.reshape(v, (1, width)).astype(jnp.float32)


def _edge_layer(lp, a_tbl, b_tbl, ea_full, ea_chunks, dst_chunks, src_chunks,
                zeros_stripe):
    full = _full_spec
    upd_chunks = []
    aggs = []
    w0 = lp["l0"]["w"]
    off = 0
    for ci, ec in enumerate(CHUNK_SIZES):
        ga, gb = _sc_gather(a_tbl, b_tbl, dst_chunks[ci], src_chunks[ci])
        if ea_full is not None:
            ea_spec = _row_spec(BE, off // BE)
            ea_arg = ea_full
        else:
            ea_spec = _row_spec(BE)
            ea_arg = ea_chunks[ci]
        edge_mlp = _tc_call(
            _edge_mlp_body, 1, ec // BE,
            [
                _row_spec(BE), _row_spec(BE), ea_spec,
                full((H, H)), full((1, H)), full((H, H)), full((1, H)),
                full((1, H)), full((1, H)),
            ],
            BE,
        )
        (upd,) = edge_mlp(
            ga, gb, ea_arg,
            w0[2 * H:], _rowvec(lp["l0"]["b"]),
            lp["l1"]["w"], _rowvec(lp["l1"]["b"]),
            _rowvec(lp["ln"]["g"]), _rowvec(lp["ln"]["b"]),
        )
        upd_chunks.append(upd)
        agg2 = _sc_scatter(upd, src_chunks[ci], zeros_stripe)
        aggs.append(agg2[0])
        aggs.append(agg2[1])
        off += ec
    return upd_chunks, aggs


def kernel(x, edge_index, edge_attr, mean_vec_x, std_vec_x, mean_vec_edge,
           std_vec_edge, params):
    dst = edge_index[1].astype(jnp.int32)
    src = edge_index[0].astype(jnp.int32)
    dst_chunks, src_chunks = [], []
    off = 0
    for ec in CHUNK_SIZES:
        dst_chunks.append(lax.slice(dst, (off,), (off + ec,)))
        src_chunks.append(lax.slice(src, (off,), (off + ec,)))
        off += ec

    pe = params["node_enc"]
    pee = params["edge_enc"]
    l0 = params["layers"][0]
    l1 = params["layers"][1]
    pd = params["dec"]

    w0e0 = l0["edge_mlp"]["l0"]["w"]
    w0e1 = l1["edge_mlp"]["l0"]["w"]

    sig_e = std_vec_edge.astype(jnp.float32)
    w0_enc = pee["l0"]["w"] / sig_e[:, None]
    b0_enc = pee["l0"]["b"] - mean_vec_edge @ w0_enc

    full = _full_spec
    node_enc = _tc_call(
        _node_enc_body, 3, N // BN,
        [
            _row_spec(BN), full((1, H)), full((1, H)),
            full((H, H)), full((1, H)), full((H, H)), full((1, H)),
            full((1, H)), full((1, H)), full((H, H)), full((H, H)),
        ],
        BN,
    )
    xe, a_tbl, b_tbl = node_enc(
        x.astype(jnp.float32), _rowvec(mean_vec_x), _rowvec(std_vec_x),
        pe["l0"]["w"], _rowvec(pe["l0"]["b"]), pe["l1"]["w"], _rowvec(pe["l1"]["b"]),
        _rowvec(pe["ln"]["g"]), _rowvec(pe["ln"]["b"]),
        w0e0[:H], w0e0[H:2 * H],
    )

    ea_lin = edge_attr.astype(jnp.float32) @ w0_enc + b0_enc
    edge_enc = _tc_call(
        _edge_enc_body, 1, E // BE,
        [
            _row_spec(BE),
            full((H, H)), full((1, H)), full((1, H)), full((1, H)),
        ],
        BE,
    )
    (ea,) = edge_enc(
        ea_lin,
        pee["l1"]["w"], _rowvec(pee["l1"]["b"]),
        _rowvec(pee["ln"]["g"]), _rowvec(pee["ln"]["b"]),
    )

    zeros_stripe = jnp.zeros((ROWS_PER_SUB, H), jnp.float32)

    node_mlp = _tc_call(
        _node_mlp_body, 3, N // BN,
        [_row_spec(BN)] + [_row_spec(BN)] * NAGG + [
            full((H, H)), full((H, H)), full((1, H)), full((H, H)), full((1, H)),
            full((1, H)), full((1, H)), full((H, H)), full((H, H)),
        ],
        BN,
    )

    upd_chunks, aggs = _edge_layer(
        l0["edge_mlp"], a_tbl, b_tbl, ea, None, dst_chunks, src_chunks, zeros_stripe)
    np0 = l0["node_mlp"]
    xe, a_tbl, b_tbl = node_mlp(
        xe, *aggs,
        np0["l0"]["w"][:H], np0["l0"]["w"][H:], _rowvec(np0["l0"]["b"]),
        np0["l1"]["w"], _rowvec(np0["l1"]["b"]),
        _rowvec(np0["ln"]["g"]), _rowvec(np0["ln"]["b"]),
        w0e1[:H], w0e1[H:2 * H],
    )

    upd_chunks, aggs = _edge_layer(
        l1["edge_mlp"], a_tbl, b_tbl, None, upd_chunks, dst_chunks, src_chunks,
        zeros_stripe)

    dw1 = jnp.zeros((H, H), jnp.float32).at[:, :DOUT].set(pd["l1"]["w"])
    db1 = jnp.zeros((H,), jnp.float32).at[:DOUT].set(pd["l1"]["b"])
    np1 = l1["node_mlp"]
    node_dec = _tc_call(
        _node_dec_body, 1, N // BN,
        [_row_spec(BN)] + [_row_spec(BN)] * NAGG + [
            full((H, H)), full((H, H)), full((1, H)), full((H, H)), full((1, H)),
            full((1, H)), full((1, H)),
            full((H, H)), full((1, H)), full((H, H)), full((1, H)),
        ],
        BN,
    )
    (out,) = node_dec(
        xe, *aggs,
        np1["l0"]["w"][:H], np1["l0"]["w"][H:], _rowvec(np1["l0"]["b"]),
        np1["l1"]["w"], _rowvec(np1["l1"]["b"]),
        _rowvec(np1["ln"]["g"]), _rowvec(np1["ln"]["b"]),
        pd["l0"]["w"], _rowvec(pd["l0"]["b"]), dw1, _rowvec(db1),
    )
    return out[:, :DOUT]


# speedup vs baseline: 5.4233x; 1.1026x over previous
import functools

import jax
import jax.numpy as jnp
from jax import lax
from jax.experimental import pallas as pl
from jax.experimental.pallas import tpu as pltpu
from jax.experimental.pallas import tpu_sc as plsc

N = 10000
E = 320000
H = 128
DOUT = 3

BE = 2560
BN = 2000

NC = 2
NS = 16
NW = NC * NS
C = 80
NPAD = 10240
ROWS_PER_SUB = NPAD // NS

CHUNK_PIECES = [41, 41, 43]
CHUNK_SIZES = [p * BE for p in CHUNK_PIECES]


def _ln(u, g, b):
    m = jnp.mean(u, axis=-1, keepdims=True)
    v = jnp.mean((u - m) ** 2, axis=-1, keepdims=True)
    return (u - m) * lax.rsqrt(v + 1e-5) * g + b


def _f32(*shape):
    return jax.ShapeDtypeStruct(shape, jnp.float32)


NAGG = 2 * len(CHUNK_SIZES)


def _node_enc_body(x, mu, sig, w0, b0, w1, b1, g, b, wa, wb, xe, a, bt):
    xn = (x[...] - mu[...]) / sig[...]
    h = jnp.maximum(jnp.dot(xn, w0[...], preferred_element_type=jnp.float32) + b0[...], 0.0)
    u = jnp.dot(h, w1[...], preferred_element_type=jnp.float32) + b1[...]
    o = _ln(u, g[...], b[...])
    xe[...] = o
    a[...] = jnp.dot(o, wa[...], preferred_element_type=jnp.float32)
    bt[...] = jnp.dot(o, wb[...], preferred_element_type=jnp.float32)


def _edge_enc_mlp_body(ga, gb, ealin, ew1, eb1, eg, ebt, w0, b0, w1, b1, g, b, out):
    ench = jnp.maximum(ealin[...], 0.0)
    encu = jnp.dot(ench, ew1[...], preferred_element_type=jnp.float32) + eb1[...]
    ea = _ln(encu, eg[...], ebt[...])
    s = ga[...] + gb[...] + jnp.dot(ea, w0[...], preferred_element_type=jnp.float32) + b0[...]
    h = jnp.maximum(s, 0.0)
    u = jnp.dot(h, w1[...], preferred_element_type=jnp.float32) + b1[...]
    out[...] = _ln(u, g[...], b[...]) + ea


def _edge_mlp_body(ga, gb, ea, w0, b0, w1, b1, g, b, out):
    s = ga[...] + gb[...] + jnp.dot(ea[...], w0[...], preferred_element_type=jnp.float32) + b0[...]
    h = jnp.maximum(s, 0.0)
    u = jnp.dot(h, w1[...], preferred_element_type=jnp.float32) + b1[...]
    out[...] = _ln(u, g[...], b[...]) + ea[...]


def _node_mlp_body(*refs):
    x = refs[0]
    aggs = refs[1:1 + NAGG]
    w0x, w0a, b0, w1, b1, g, b, wa, wb = refs[1 + NAGG:10 + NAGG]
    xo, a, bt = refs[10 + NAGG:]
    agg = aggs[0][...]
    for r in aggs[1:]:
        agg = agg + r[...]
    s = (jnp.dot(x[...], w0x[...], preferred_element_type=jnp.float32)
         + jnp.dot(agg, w0a[...], preferred_element_type=jnp.float32) + b0[...])
    h = jnp.maximum(s, 0.0)
    u = jnp.dot(h, w1[...], preferred_element_type=jnp.float32) + b1[...]
    xn = x[...] + _ln(u, g[...], b[...])
    xo[...] = xn
    a[...] = jnp.dot(xn, wa[...], preferred_element_type=jnp.float32)
    bt[...] = jnp.dot(xn, wb[...], preferred_element_type=jnp.float32)


def _node_dec_body(*refs):
    x = refs[0]
    aggs = refs[1:1 + NAGG]
    w0x, w0a, b0, w1, b1, g, b, dw0, db0, dw1, db1 = refs[1 + NAGG:12 + NAGG]
    out = refs[12 + NAGG]
    agg = aggs[0][...]
    for r in aggs[1:]:
        agg = agg + r[...]
    s = (jnp.dot(x[...], w0x[...], preferred_element_type=jnp.float32)
         + jnp.dot(agg, w0a[...], preferred_element_type=jnp.float32) + b0[...])
    h = jnp.maximum(s, 0.0)
    u = jnp.dot(h, w1[...], preferred_element_type=jnp.float32) + b1[...]
    xn = x[...] + _ln(u, g[...], b[...])
    dh = jnp.maximum(jnp.dot(xn, dw0[...], preferred_element_type=jnp.float32) + db0[...], 0.0)
    out[...] = jnp.dot(dh, dw1[...], preferred_element_type=jnp.float32) + db1[...]


def _row_spec(rows, off_blocks=0):
    if off_blocks:
        return pl.BlockSpec((rows, H), lambda i: (i + off_blocks, 0))
    return pl.BlockSpec((rows, H), lambda i: (i, 0))


def _full_spec(shape):
    nd = len(shape)
    return pl.BlockSpec(shape, lambda i: (0,) * nd)


def _tc_call(body, n_out, grid, in_specs, out_rows, interpret=False):
    return pl.pallas_call(
        body,
        grid=(grid,),
        in_specs=in_specs,
        out_specs=[_row_spec(out_rows)] * n_out,
        out_shape=[_f32(grid * out_rows, H)] * n_out,
        interpret=interpret,
    )


def _sc_gather(a_tbl, b_tbl, dst, src):
    ec = dst.shape[0]
    epw = ec // NW
    nchunk = epw // C
    assert nchunk % 2 == 1 and nchunk * C == epw
    mesh = plsc.VectorSubcoreMesh(core_axis_name="c", subcore_axis_name="s")

    @functools.partial(
        pl.kernel,
        out_type=[_f32(ec, H), _f32(ec, H)],
        mesh=mesh,
        scratch_types=[
            pltpu.VMEM((epw,), jnp.int32),
            pltpu.VMEM((epw,), jnp.int32),
            pltpu.VMEM((C, H), jnp.float32),
            pltpu.VMEM((C, H), jnp.float32),
            pltpu.VMEM((C, H), jnp.float32),
            pltpu.VMEM((C, H), jnp.float32),
            pltpu.SemaphoreType.DMA,
            pltpu.SemaphoreType.DMA,
            pltpu.SemaphoreType.DMA,
            pltpu.SemaphoreType.DMA,
            pltpu.SemaphoreType.DMA,
            pltpu.SemaphoreType.DMA,
            pltpu.SemaphoreType.DMA,
            pltpu.SemaphoreType.DMA,
        ],
    )
    def k(a_hbm, b_hbm, dst_hbm, src_hbm, ga_hbm, gb_hbm,
          didx, sidx, a0, b0, a1, b1,
          sga0, sgb0, sga1, sgb1, wa0, wb0, wa1, wb1):
        wid = lax.axis_index("s") * NC + lax.axis_index("c")
        base = wid * epw
        pltpu.sync_copy(dst_hbm.at[pl.ds(base, epw)], didx)
        pltpu.sync_copy(src_hbm.at[pl.ds(base, epw)], sidx)

        def gather(c, buf_a, buf_b, sa, sb):
            pltpu.async_copy(a_hbm.at[didx.at[pl.ds(c * C, C)]], buf_a, sa)
            pltpu.async_copy(b_hbm.at[sidx.at[pl.ds(c * C, C)]], buf_b, sb)

        def write(c, buf_a, buf_b, sa, sb):
            pltpu.async_copy(buf_a, ga_hbm.at[pl.ds(base + c * C, C)], sa)
            pltpu.async_copy(buf_b, gb_hbm.at[pl.ds(base + c * C, C)], sb)

        def drain_write(buf_a, buf_b, sa, sb):
            pltpu.make_async_copy(buf_a, ga_hbm.at[pl.ds(base, C)], sa).wait()
            pltpu.make_async_copy(buf_b, gb_hbm.at[pl.ds(base, C)], sb).wait()

        def wait_gather(buf_a, buf_b, sa, sb):
            pltpu.make_async_copy(a_hbm.at[didx.at[pl.ds(0, C)]], buf_a, sa).wait()
            pltpu.make_async_copy(b_hbm.at[sidx.at[pl.ds(0, C)]], buf_b, sb).wait()

        gather(0, a0, b0, sga0, sgb0)

        def step(i, _):
            @pl.when(i > 0)
            def _():
                drain_write(a1, b1, wa1, wb1)
            gather(2 * i + 1, a1, b1, sga1, sgb1)
            wait_gather(a0, b0, sga0, sgb0)
            write(2 * i, a0, b0, wa0, wb0)
            drain_write(a0, b0, wa0, wb0)
            gather(2 * i + 2, a0, b0, sga0, sgb0)
            wait_gather(a1, b1, sga1, sgb1)
            write(2 * i + 1, a1, b1, wa1, wb1)
            return 0

        lax.fori_loop(0, (nchunk - 1) // 2, step, 0)
        drain_write(a1, b1, wa1, wb1)
        wait_gather(a0, b0, sga0, sgb0)
        write(nchunk - 1, a0, b0, wa0, wb0)
        drain_write(a0, b0, wa0, wb0)

    return k(a_tbl, b_tbl, dst, src)


def _sc_scatter(upd, src, zeros_stripe):
    ec = src.shape[0]
    epw = ec // NW
    nchunk = epw // C
    assert nchunk % 2 == 1 and nchunk * C == epw
    mesh = plsc.VectorSubcoreMesh(core_axis_name="c", subcore_axis_name="s")

    @functools.partial(
        pl.kernel,
        out_type=_f32(NC, NPAD, H),
        mesh=mesh,
        scratch_types=[
            pltpu.VMEM((epw,), jnp.int32),
            pltpu.VMEM((C, H), jnp.float32),
            pltpu.VMEM((C, H), jnp.float32),
            pltpu.SemaphoreType.DMA,
            pltpu.SemaphoreType.DMA,
            pltpu.VMEM_SHARED((NPAD, H), jnp.float32),
        ],
    )
    def k(upd_hbm, src_hbm, z_hbm, agg_hbm, sidx, buf0, buf1, r0, r1, shared):
        cid = lax.axis_index("c")
        sid = lax.axis_index("s")
        wid = sid * NC + cid
        base = wid * epw
        stripe = sid * ROWS_PER_SUB
        pltpu.sync_copy(z_hbm, shared.at[pl.ds(stripe, ROWS_PER_SUB)])
        pltpu.sync_copy(src_hbm.at[pl.ds(base, epw)], sidx)
        plsc.subcore_barrier()

        def read(c, buf, sem):
            pltpu.async_copy(upd_hbm.at[pl.ds(base + c * C, C)], buf, sem)

        def wait_read(buf, sem):
            pltpu.make_async_copy(upd_hbm.at[pl.ds(base, C)], buf, sem).wait()

        def scat(c, buf):
            pltpu.sync_copy(buf, shared.at[sidx.at[pl.ds(c * C, C)]], add=True)

        read(0, buf0, r0)

        def step(i, _):
            read(2 * i + 1, buf1, r1)
            wait_read(buf0, r0)
            scat(2 * i, buf0)
            read(2 * i + 2, buf0, r0)
            wait_read(buf1, r1)
            scat(2 * i + 1, buf1)
            return 0

        lax.fori_loop(0, (nchunk - 1) // 2, step, 0)
        wait_read(buf0, r0)
        scat(nchunk - 1, buf0)
        plsc.subcore_barrier()
        pltpu.sync_copy(shared.at[pl.ds(stripe, ROWS_PER_SUB)],
                        agg_hbm.at[cid, pl.ds(stripe, ROWS_PER_SUB)])

    return k(upd, src, zeros_stripe)


def _rowvec(v, width=H):
    return jnp.reshape(v, (1, width)).astype(jnp.float32)


def _edge_layer(lp, a_tbl, b_tbl, ea_chunks, enc, dst_chunks, src_chunks,
                zeros_stripe):
    full = _full_spec
    upd_chunks = []
    aggs = []
    w0 = lp["l0"]["w"]
    mlp_args = (
        w0[2 * H:], _rowvec(lp["l0"]["b"]),
        lp["l1"]["w"], _rowvec(lp["l1"]["b"]),
        _rowvec(lp["ln"]["g"]), _rowvec(lp["ln"]["b"]),
    )
    mlp_specs = [
        full((H, H)), full((1, H)), full((H, H)), full((1, H)),
        full((1, H)), full((1, H)),
    ]
    for ci, ec in enumerate(CHUNK_SIZES):
        ga, gb = _sc_gather(a_tbl, b_tbl, dst_chunks[ci], src_chunks[ci])
        if enc is not None:
            body = _edge_enc_mlp_body
            specs = ([_row_spec(BE)] * 3
                     + [full((H, H)), full((1, H)), full((1, H)), full((1, H))]
                     + mlp_specs)
            args = (ga, gb, ea_chunks[ci]) + enc + mlp_args
        else:
            body = _edge_mlp_body
            specs = [_row_spec(BE)] * 3 + mlp_specs
            args = (ga, gb, ea_chunks[ci]) + mlp_args
        edge_mlp = _tc_call(body, 1, ec // BE, specs, BE)
        (upd,) = edge_mlp(*args)
        upd_chunks.append(upd)
        agg2 = _sc_scatter(upd, src_chunks[ci], zeros_stripe)
        aggs.append(agg2[0])
        aggs.append(agg2[1])
    return upd_chunks, aggs


def kernel(x, edge_index, edge_attr, mean_vec_x, std_vec_x, mean_vec_edge,
           std_vec_edge, params):
    dst = edge_index[1].astype(jnp.int32)
    src = edge_index[0].astype(jnp.int32)
    dst_chunks, src_chunks = [], []
    off = 0
    for ec in CHUNK_SIZES:
        dst_chunks.append(lax.slice(dst, (off,), (off + ec,)))
        src_chunks.append(lax.slice(src, (off,), (off + ec,)))
        off += ec

    pe = params["node_enc"]
    pee = params["edge_enc"]
    l0 = params["layers"][0]
    l1 = params["layers"][1]
    pd = params["dec"]

    w0e0 = l0["edge_mlp"]["l0"]["w"]
    w0e1 = l1["edge_mlp"]["l0"]["w"]

    sig_e = std_vec_edge.astype(jnp.float32)
    w0_enc = pee["l0"]["w"] / sig_e[:, None]
    b0_enc = pee["l0"]["b"] - mean_vec_edge @ w0_enc

    full = _full_spec
    node_enc = _tc_call(
        _node_enc_body, 3, N // BN,
        [
            _row_spec(BN), full((1, H)), full((1, H)),
            full((H, H)), full((1, H)), full((H, H)), full((1, H)),
            full((1, H)), full((1, H)), full((H, H)), full((H, H)),
        ],
        BN,
    )
    xe, a_tbl, b_tbl = node_enc(
        x.astype(jnp.float32), _rowvec(mean_vec_x), _rowvec(std_vec_x),
        pe["l0"]["w"], _rowvec(pe["l0"]["b"]), pe["l1"]["w"], _rowvec(pe["l1"]["b"]),
        _rowvec(pe["ln"]["g"]), _rowvec(pe["ln"]["b"]),
        w0e0[:H], w0e0[H:2 * H],
    )

    ea_f32 = edge_attr.astype(jnp.float32)
    ea_lin_chunks = []
    off = 0
    for ec in CHUNK_SIZES:
        ea_lin_chunks.append(
            lax.slice(ea_f32, (off, 0), (off + ec, 4)) @ w0_enc + b0_enc)
        off += ec
    enc = (pee["l1"]["w"], _rowvec(pee["l1"]["b"]),
           _rowvec(pee["ln"]["g"]), _rowvec(pee["ln"]["b"]))

    zeros_stripe = jnp.zeros((ROWS_PER_SUB, H), jnp.float32)

    node_mlp = _tc_call(
        _node_mlp_body, 3, N // BN,
        [_row_spec(BN)] + [_row_spec(BN)] * NAGG + [
            full((H, H)), full((H, H)), full((1, H)), full((H, H)), full((1, H)),
            full((1, H)), full((1, H)), full((H, H)), full((H, H)),
        ],
        BN,
    )

    upd_chunks, aggs = _edge_layer(
        l0["edge_mlp"], a_tbl, b_tbl, ea_lin_chunks, enc, dst_chunks, src_chunks,
        zeros_stripe)
    np0 = l0["node_mlp"]
    xe, a_tbl, b_tbl = node_mlp(
        xe, *aggs,
        np0["l0"]["w"][:H], np0["l0"]["w"][H:], _rowvec(np0["l0"]["b"]),
        np0["l1"]["w"], _rowvec(np0["l1"]["b"]),
        _rowvec(np0["ln"]["g"]), _rowvec(np0["ln"]["b"]),
        w0e1[:H], w0e1[H:2 * H],
    )

    upd_chunks, aggs = _edge_layer(
        l1["edge_mlp"], a_tbl, b_tbl, upd_chunks, None, dst_chunks, src_chunks,
        zeros_stripe)

    dw1 = jnp.zeros((H, H), jnp.float32).at[:, :DOUT].set(pd["l1"]["w"])
    db1 = jnp.zeros((H,), jnp.float32).at[:DOUT].set(pd["l1"]["b"])
    np1 = l1["node_mlp"]
    node_dec = _tc_call(
        _node_dec_body, 1, N // BN,
        [_row_spec(BN)] + [_row_spec(BN)] * NAGG + [
            full((H, H)), full((H, H)), full((1, H)), full((H, H)), full((1, H)),
            full((1, H)), full((1, H)),
            full((H, H)), full((1, H)), full((H, H)), full((1, H)),
        ],
        BN,
    )
    (out,) = node_dec(
        xe, *aggs,
        np1["l0"]["w"][:H], np1["l0"]["w"][H:], _rowvec(np1["l0"]["b"]),
        np1["l1"]["w"], _rowvec(np1["l1"]["b"]),
        _rowvec(np1["ln"]["g"]), _rowvec(np1["ln"]["b"]),
        pd["l0"]["w"], _rowvec(pd["l0"]["b"]), dw1, _rowvec(db1),
    )
    return out[:, :DOUT]


# speedup vs baseline: 6.2468x; 1.1518x over previous
import functools

import jax
import jax.numpy as jnp
from jax import lax
from jax.experimental import pallas as pl
from jax.experimental.pallas import tpu as pltpu
from jax.experimental.pallas import tpu_sc as plsc

N = 10000
E = 320000
H = 128
DOUT = 3

BE = 2560
BN = 2000

NC = 2
NS = 16
NW = NC * NS
C = 80
NPAD = 10240
ROWS_PER_SUB = NPAD // NS

CHUNK_PIECES = [41, 41, 43]
CHUNK_SIZES = [p * BE for p in CHUNK_PIECES]


def _ln(u, g, b):
    m = jnp.mean(u, axis=-1, keepdims=True)
    v = jnp.mean((u - m) ** 2, axis=-1, keepdims=True)
    return (u - m) * lax.rsqrt(v + 1e-5) * g + b


def _f32(*shape):
    return jax.ShapeDtypeStruct(shape, jnp.float32)


NAGG = 2 * len(CHUNK_SIZES)


def _node_enc_body(x, mu, sig, w0, b0, w1, b1, g, b, wa, wb, xe, a, bt):
    xn = (x[...] - mu[...]) / sig[...]
    h = jnp.maximum(jnp.dot(xn, w0[...], preferred_element_type=jnp.float32) + b0[...], 0.0)
    u = jnp.dot(h, w1[...], preferred_element_type=jnp.float32) + b1[...]
    o = _ln(u, g[...], b[...])
    xe[...] = o
    a[...] = jnp.dot(o, wa[...], preferred_element_type=jnp.float32)
    bt[...] = jnp.dot(o, wb[...], preferred_element_type=jnp.float32)


def _edge_enc_mlp_body(gsum, ealin, ew1, eb1, eg, ebt, w0, b0, w1, b1, g, b, out):
    ench = jnp.maximum(ealin[...], 0.0)
    encu = jnp.dot(ench, ew1[...], preferred_element_type=jnp.float32) + eb1[...]
    ea = _ln(encu, eg[...], ebt[...])
    s = gsum[...] + jnp.dot(ea, w0[...], preferred_element_type=jnp.float32) + b0[...]
    h = jnp.maximum(s, 0.0)
    u = jnp.dot(h, w1[...], preferred_element_type=jnp.float32) + b1[...]
    out[...] = _ln(u, g[...], b[...]) + ea


def _edge_mlp_body(gsum, ea, w0, b0, w1, b1, g, b, out):
    s = gsum[...] + jnp.dot(ea[...], w0[...], preferred_element_type=jnp.float32) + b0[...]
    h = jnp.maximum(s, 0.0)
    u = jnp.dot(h, w1[...], preferred_element_type=jnp.float32) + b1[...]
    out[...] = _ln(u, g[...], b[...]) + ea[...]


def _node_mlp_body(*refs):
    x = refs[0]
    aggs = refs[1:1 + NAGG]
    w0x, w0a, b0, w1, b1, g, b, wa, wb = refs[1 + NAGG:10 + NAGG]
    xo, a, bt = refs[10 + NAGG:]
    agg = aggs[0][...]
    for r in aggs[1:]:
        agg = agg + r[...]
    s = (jnp.dot(x[...], w0x[...], preferred_element_type=jnp.float32)
         + jnp.dot(agg, w0a[...], preferred_element_type=jnp.float32) + b0[...])
    h = jnp.maximum(s, 0.0)
    u = jnp.dot(h, w1[...], preferred_element_type=jnp.float32) + b1[...]
    xn = x[...] + _ln(u, g[...], b[...])
    xo[...] = xn
    a[...] = jnp.dot(xn, wa[...], preferred_element_type=jnp.float32)
    bt[...] = jnp.dot(xn, wb[...], preferred_element_type=jnp.float32)


def _node_dec_body(*refs):
    x = refs[0]
    aggs = refs[1:1 + NAGG]
    w0x, w0a, b0, w1, b1, g, b, dw0, db0, dw1, db1 = refs[1 + NAGG:12 + NAGG]
    out = refs[12 + NAGG]
    agg = aggs[0][...]
    for r in aggs[1:]:
        agg = agg + r[...]
    s = (jnp.dot(x[...], w0x[...], preferred_element_type=jnp.float32)
         + jnp.dot(agg, w0a[...], preferred_element_type=jnp.float32) + b0[...])
    h = jnp.maximum(s, 0.0)
    u = jnp.dot(h, w1[...], preferred_element_type=jnp.float32) + b1[...]
    xn = x[...] + _ln(u, g[...], b[...])
    dh = jnp.maximum(jnp.dot(xn, dw0[...], preferred_element_type=jnp.float32) + db0[...], 0.0)
    out[...] = jnp.dot(dh, dw1[...], preferred_element_type=jnp.float32) + db1[...]


def _row_spec(rows, off_blocks=0):
    if off_blocks:
        return pl.BlockSpec((rows, H), lambda i: (i + off_blocks, 0))
    return pl.BlockSpec((rows, H), lambda i: (i, 0))


def _full_spec(shape):
    nd = len(shape)
    return pl.BlockSpec(shape, lambda i: (0,) * nd)


def _tc_call(body, n_out, grid, in_specs, out_rows, interpret=False):
    return pl.pallas_call(
        body,
        grid=(grid,),
        in_specs=in_specs,
        out_specs=[_row_spec(out_rows)] * n_out,
        out_shape=[_f32(grid * out_rows, H)] * n_out,
        interpret=interpret,
    )


def _sc_gather(a_tbl, b_tbl, dst, src):
    ec = dst.shape[0]
    epw = ec // NW
    nchunk = epw // C
    assert nchunk % 2 == 1 and nchunk * C == epw
    mesh = plsc.VectorSubcoreMesh(core_axis_name="c", subcore_axis_name="s")

    @functools.partial(
        pl.kernel,
        out_type=_f32(ec, H),
        mesh=mesh,
        scratch_types=[
            pltpu.VMEM((epw,), jnp.int32),
            pltpu.VMEM((epw,), jnp.int32),
            pltpu.VMEM((C, H), jnp.float32),
            pltpu.VMEM((C, H), jnp.float32),
            pltpu.VMEM((C, H), jnp.float32),
            pltpu.VMEM((C, H), jnp.float32),
            pltpu.SemaphoreType.DMA,
            pltpu.SemaphoreType.DMA,
            pltpu.SemaphoreType.DMA,
            pltpu.SemaphoreType.DMA,
            pltpu.SemaphoreType.DMA,
            pltpu.SemaphoreType.DMA,
        ],
    )
    def k(a_hbm, b_hbm, dst_hbm, src_hbm, g_hbm,
          didx, sidx, a0, b0, a1, b1,
          sga0, sgb0, sga1, sgb1, wa0, wa1):
        wid = lax.axis_index("s") * NC + lax.axis_index("c")
        base = wid * epw
        pltpu.sync_copy(dst_hbm.at[pl.ds(base, epw)], didx)
        pltpu.sync_copy(src_hbm.at[pl.ds(base, epw)], sidx)

        def gather(c, buf_a, buf_b, sa, sb):
            pltpu.async_copy(a_hbm.at[didx.at[pl.ds(c * C, C)]], buf_a, sa)
            pltpu.async_copy(b_hbm.at[sidx.at[pl.ds(c * C, C)]], buf_b, sb)

        def write(c, buf_a, sa):
            pltpu.async_copy(buf_a, g_hbm.at[pl.ds(base + c * C, C)], sa)

        def drain_write(buf_a, sa):
            pltpu.make_async_copy(buf_a, g_hbm.at[pl.ds(base, C)], sa).wait()

        def wait_gather(buf_a, buf_b, sa, sb):
            pltpu.make_async_copy(a_hbm.at[didx.at[pl.ds(0, C)]], buf_a, sa).wait()
            pltpu.make_async_copy(b_hbm.at[sidx.at[pl.ds(0, C)]], buf_b, sb).wait()

        def addinto(buf_a, buf_b):
            def row(r, _):
                for j in range(H // 16):
                    sl = pl.ds(j * 16, 16)
                    buf_a[r, sl] = buf_a[r, sl] + buf_b[r, sl]
                return 0
            lax.fori_loop(0, C, row, 0)

        gather(0, a0, b0, sga0, sgb0)

        def step(i, _):
            @pl.when(i > 0)
            def _():
                drain_write(a1, wa1)
            gather(2 * i + 1, a1, b1, sga1, sgb1)
            wait_gather(a0, b0, sga0, sgb0)
            addinto(a0, b0)
            write(2 * i, a0, wa0)
            drain_write(a0, wa0)
            gather(2 * i + 2, a0, b0, sga0, sgb0)
            wait_gather(a1, b1, sga1, sgb1)
            addinto(a1, b1)
            write(2 * i + 1, a1, wa1)
            return 0

        lax.fori_loop(0, (nchunk - 1) // 2, step, 0)
        drain_write(a1, wa1)
        wait_gather(a0, b0, sga0, sgb0)
        addinto(a0, b0)
        write(nchunk - 1, a0, wa0)
        drain_write(a0, wa0)

    return k(a_tbl, b_tbl, dst, src)


def _sc_scatter(upd, src, zeros_stripe):
    ec = src.shape[0]
    epw = ec // NW
    nchunk = epw // C
    assert nchunk % 2 == 1 and nchunk * C == epw
    mesh = plsc.VectorSubcoreMesh(core_axis_name="c", subcore_axis_name="s")

    @functools.partial(
        pl.kernel,
        out_type=_f32(NC, NPAD, H),
        mesh=mesh,
        scratch_types=[
            pltpu.VMEM((epw,), jnp.int32),
            pltpu.VMEM((C, H), jnp.float32),
            pltpu.VMEM((C, H), jnp.float32),
            pltpu.SemaphoreType.DMA,
            pltpu.SemaphoreType.DMA,
            pltpu.VMEM_SHARED((NPAD, H), jnp.float32),
        ],
    )
    def k(upd_hbm, src_hbm, z_hbm, agg_hbm, sidx, buf0, buf1, r0, r1, shared):
        cid = lax.axis_index("c")
        sid = lax.axis_index("s")
        wid = sid * NC + cid
        base = wid * epw
        stripe = sid * ROWS_PER_SUB
        pltpu.sync_copy(z_hbm, shared.at[pl.ds(stripe, ROWS_PER_SUB)])
        pltpu.sync_copy(src_hbm.at[pl.ds(base, epw)], sidx)
        plsc.subcore_barrier()

        def read(c, buf, sem):
            pltpu.async_copy(upd_hbm.at[pl.ds(base + c * C, C)], buf, sem)

        def wait_read(buf, sem):
            pltpu.make_async_copy(upd_hbm.at[pl.ds(base, C)], buf, sem).wait()

        def scat(c, buf):
            pltpu.sync_copy(buf, shared.at[sidx.at[pl.ds(c * C, C)]], add=True)

        read(0, buf0, r0)

        def step(i, _):
            read(2 * i + 1, buf1, r1)
            wait_read(buf0, r0)
            scat(2 * i, buf0)
            read(2 * i + 2, buf0, r0)
            wait_read(buf1, r1)
            scat(2 * i + 1, buf1)
            return 0

        lax.fori_loop(0, (nchunk - 1) // 2, step, 0)
        wait_read(buf0, r0)
        scat(nchunk - 1, buf0)
        plsc.subcore_barrier()
        pltpu.sync_copy(shared.at[pl.ds(stripe, ROWS_PER_SUB)],
                        agg_hbm.at[cid, pl.ds(stripe, ROWS_PER_SUB)])

    return k(upd, src, zeros_stripe)


def _rowvec(v, width=H):
    return jnp.reshape(v, (1, width)).astype(jnp.float32)


def _edge_layer(lp, a_tbl, b_tbl, ea_chunks, enc, dst_chunks, src_chunks,
                zeros_stripe):
    full = _full_spec
    upd_chunks = []
    aggs = []
    w0 = lp["l0"]["w"]
    mlp_args = (
        w0[2 * H:], _rowvec(lp["l0"]["b"]),
        lp["l1"]["w"], _rowvec(lp["l1"]["b"]),
        _rowvec(lp["ln"]["g"]), _rowvec(lp["ln"]["b"]),
    )
    mlp_specs = [
        full((H, H)), full((1, H)), full((H, H)), full((1, H)),
        full((1, H)), full((1, H)),
    ]
    for ci, ec in enumerate(CHUNK_SIZES):
        gsum = _sc_gather(a_tbl, b_tbl, dst_chunks[ci], src_chunks[ci])
        if enc is not None:
            body = _edge_enc_mlp_body
            specs = ([_row_spec(BE)] * 2
                     + [full((H, H)), full((1, H)), full((1, H)), full((1, H))]
                     + mlp_specs)
            args = (gsum, ea_chunks[ci]) + enc + mlp_args
        else:
            body = _edge_mlp_body
            specs = [_row_spec(BE)] * 2 + mlp_specs
            args = (gsum, ea_chunks[ci]) + mlp_args
        edge_mlp = _tc_call(body, 1, ec // BE, specs, BE)
        (upd,) = edge_mlp(*args)
        upd_chunks.append(upd)
        agg2 = _sc_scatter(upd, src_chunks[ci], zeros_stripe)
        aggs.append(agg2[0])
        aggs.append(agg2[1])
    return upd_chunks, aggs


def kernel(x, edge_index, edge_attr, mean_vec_x, std_vec_x, mean_vec_edge,
           std_vec_edge, params):
    dst = edge_index[1].astype(jnp.int32)
    src = edge_index[0].astype(jnp.int32)
    dst_chunks, src_chunks = [], []
    off = 0
    for ec in CHUNK_SIZES:
        dst_chunks.append(lax.slice(dst, (off,), (off + ec,)))
        src_chunks.append(lax.slice(src, (off,), (off + ec,)))
        off += ec

    pe = params["node_enc"]
    pee = params["edge_enc"]
    l0 = params["layers"][0]
    l1 = params["layers"][1]
    pd = params["dec"]

    w0e0 = l0["edge_mlp"]["l0"]["w"]
    w0e1 = l1["edge_mlp"]["l0"]["w"]

    sig_e = std_vec_edge.astype(jnp.float32)
    w0_enc = pee["l0"]["w"] / sig_e[:, None]
    b0_enc = pee["l0"]["b"] - mean_vec_edge @ w0_enc

    full = _full_spec
    node_enc = _tc_call(
        _node_enc_body, 3, N // BN,
        [
            _row_spec(BN), full((1, H)), full((1, H)),
            full((H, H)), full((1, H)), full((H, H)), full((1, H)),
            full((1, H)), full((1, H)), full((H, H)), full((H, H)),
        ],
        BN,
    )
    xe, a_tbl, b_tbl = node_enc(
        x.astype(jnp.float32), _rowvec(mean_vec_x), _rowvec(std_vec_x),
        pe["l0"]["w"], _rowvec(pe["l0"]["b"]), pe["l1"]["w"], _rowvec(pe["l1"]["b"]),
        _rowvec(pe["ln"]["g"]), _rowvec(pe["ln"]["b"]),
        w0e0[:H], w0e0[H:2 * H],
    )

    ea_f32 = edge_attr.astype(jnp.float32)
    ea_lin_chunks = []
    off = 0
    for ec in CHUNK_SIZES:
        ea_lin_chunks.append(
            lax.slice(ea_f32, (off, 0), (off + ec, 4)) @ w0_enc + b0_enc)
        off += ec
    enc = (pee["l1"]["w"], _rowvec(pee["l1"]["b"]),
           _rowvec(pee["ln"]["g"]), _rowvec(pee["ln"]["b"]))

    zeros_stripe = jnp.zeros((ROWS_PER_SUB, H), jnp.float32)

    node_mlp = _tc_call(
        _node_mlp_body, 3, N // BN,
        [_row_spec(BN)] + [_row_spec(BN)] * NAGG + [
            full((H, H)), full((H, H)), full((1, H)), full((H, H)), full((1, H)),
            full((1, H)), full((1, H)), full((H, H)), full((H, H)),
        ],
        BN,
    )

    upd_chunks, aggs = _edge_layer(
        l0["edge_mlp"], a_tbl, b_tbl, ea_lin_chunks, enc, dst_chunks, src_chunks,
        zeros_stripe)
    np0 = l0["node_mlp"]
    xe, a_tbl, b_tbl = node_mlp(
        xe, *aggs,
        np0["l0"]["w"][:H], np0["l0"]["w"][H:], _rowvec(np0["l0"]["b"]),
        np0["l1"]["w"], _rowvec(np0["l1"]["b"]),
        _rowvec(np0["ln"]["g"]), _rowvec(np0["ln"]["b"]),
        w0e1[:H], w0e1[H:2 * H],
    )

    upd_chunks, aggs = _edge_layer(
        l1["edge_mlp"], a_tbl, b_tbl, upd_chunks, None, dst_chunks, src_chunks,
        zeros_stripe)

    dw1 = jnp.zeros((H, H), jnp.float32).at[:, :DOUT].set(pd["l1"]["w"])
    db1 = jnp.zeros((H,), jnp.float32).at[:DOUT].set(pd["l1"]["b"])
    np1 = l1["node_mlp"]
    node_dec = _tc_call(
        _node_dec_body, 1, N // BN,
        [_row_spec(BN)] + [_row_spec(BN)] * NAGG + [
            full((H, H)), full((H, H)), full((1, H)), full((H, H)), full((1, H)),
            full((1, H)), full((1, H)),
            full((H, H)), full((1, H)), full((H, H)), full((1, H)),
        ],
        BN,
    )
    (out,) = node_dec(
        xe, *aggs,
        np1["l0"]["w"][:H], np1["l0"]["w"][H:], _rowvec(np1["l0"]["b"]),
        np1["l1"]["w"], _rowvec(np1["l1"]["b"]),
        _rowvec(np1["ln"]["g"]), _rowvec(np1["ln"]["b"]),
        pd["l0"]["w"], _rowvec(pd["l0"]["b"]), dw1, _rowvec(db1),
    )
    return out[:, :DOUT]
